# baseline (speedup 1.0000x reference)
import functools

import numpy as np
import jax
import jax.numpy as jnp
from jax import lax
from jax.experimental import pallas as pl
from jax.experimental.pallas import tpu as pltpu
from jax.experimental.pallas import tpu_sc as plsc

N = 10000
E = 90000
C = 14
HID = 128
NCLS = 25
NL = 3
ROUNDS = 3

XW = 48
NPAIR = C * (C + 1) // 2
PW = 128
AXW = HID + 128

CH = 112
CPTG = 26
CPTS = 52
EP = 32 * CH * CPTG
NT = 10240
STRIPE = NT // 16

ET = 1024
NTC = 2000

_f32 = jnp.float32


def _mm(a, b):
    return lax.dot_general(a, b, (((a.ndim - 1,), (0,)), ((), ())),
                           preferred_element_type=_f32)


def _silu(x):
    y = 0.5 * x
    return y + y * jnp.tanh(y)


def _np_expand_consts():
    pairs = [(c, d) for c in range(C) for d in range(c, C)]
    ru = np.zeros((XW, 3 * PW), np.float32)
    rv = np.zeros((XW, 3 * PW), np.float32)
    for p, (c, d) in enumerate(pairs):
        for i in range(3):
            ru[c * 3 + i, i * PW + p] = 1.0
            rv[d * 3 + i, i * PW + p] = 1.0
    re = np.zeros((16, XW), np.float32)
    for c in range(C):
        for i in range(3):
            re[c, c * 3 + i] = 1.0
    idx_cd = np.array([c * C + d for (c, d) in pairs], np.int32)
    idx_dc = np.array([d * C + c for (c, d) in pairs], np.int32)
    offd = np.array([1.0 if c != d else 0.0 for (c, d) in pairs], np.float32)
    return ru, rv, re, idx_cd, idx_dc, offd


_RU_NP, _RV_NP, _RE_NP, _ICD_NP, _IDC_NP, _OFFD_NP = _np_expand_consts()


@functools.cache
def _sc_mesh():
    return plsc.VectorSubcoreMesh(core_axis_name="c", subcore_axis_name="s")


def _sc_gather(AX, BX, srcg, dstg):
    @functools.partial(
        pl.kernel,
        out_type=[
            jax.ShapeDtypeStruct((EP, AXW), _f32),
            jax.ShapeDtypeStruct((EP, AXW), _f32),
        ],
        mesh=_sc_mesh(),
        scratch_types=[
            pltpu.VMEM((CPTG, CH), jnp.int32),
            pltpu.VMEM((CPTG, CH), jnp.int32),
            pltpu.VMEM((CH, AXW), _f32),
            pltpu.VMEM((CH, AXW), _f32),
            pltpu.VMEM((CH, AXW), _f32),
            pltpu.VMEM((CH, AXW), _f32),
            pltpu.SemaphoreType.DMA,
            pltpu.SemaphoreType.DMA,
            pltpu.SemaphoreType.DMA,
            pltpu.SemaphoreType.DMA,
        ],
        name="sc_gather",
    )
    def k(ax_h, bx_h, srcg_h, dstg_h, oa, ob,
          isrc, idst, ba0, bb0, ba1, bb1, gs0, gs1, ws0, ws1):
        w = lax.axis_index("c") * 16 + lax.axis_index("s")
        pltpu.sync_copy(srcg_h.at[w], isrc)
        pltpu.sync_copy(dstg_h.at[w], idst)

        def gath(j, pa, pb, gs):
            pltpu.async_copy(ax_h.at[isrc.at[j]], pa, gs)
            pltpu.async_copy(bx_h.at[idst.at[j]], pb, gs)

        def gwait(pa, pb, gs):
            pltpu.make_async_copy(ax_h.at[isrc.at[0]], pa, gs).wait()
            pltpu.make_async_copy(bx_h.at[idst.at[0]], pb, gs).wait()

        def wrt(j, pa, pb, ws):
            e0 = (w * CPTG + j) * CH
            pltpu.async_copy(pa, oa.at[pl.ds(e0, CH)], ws)
            pltpu.async_copy(pb, ob.at[pl.ds(e0, CH)], ws)

        def wwait(pa, pb, ws):
            pltpu.make_async_copy(pa, oa.at[pl.ds(0, CH)], ws).wait()
            pltpu.make_async_copy(pb, ob.at[pl.ds(0, CH)], ws).wait()

        gath(0, ba0, bb0, gs0)
        gath(1, ba1, bb1, gs1)

        @pl.loop(0, CPTG, step=2)
        def body(j0):
            gwait(ba0, bb0, gs0)
            wrt(j0, ba0, bb0, ws0)
            gwait(ba1, bb1, gs1)
            wrt(j0 + 1, ba1, bb1, ws1)
            wwait(ba0, bb0, ws0)

            @pl.when(j0 + 2 < CPTG)
            def _():
                gath(j0 + 2, ba0, bb0, gs0)

            wwait(ba1, bb1, ws1)

            @pl.when(j0 + 3 < CPTG)
            def _():
                gath(j0 + 3, ba1, bb1, gs1)

    return k(AX, BX, srcg, dstg)


def _sc_scatter(Mv, WX, dsts):
    @functools.partial(
        pl.kernel,
        out_type=[
            jax.ShapeDtypeStruct((NT, HID), _f32),
            jax.ShapeDtypeStruct((NT, HID), _f32),
        ],
        mesh=_sc_mesh(),
        scratch_types=[
            pltpu.VMEM_SHARED((NT, HID), _f32),
            pltpu.VMEM((CPTS, CH), jnp.int32),
            pltpu.VMEM((CH, HID), _f32),
            pltpu.VMEM((CH, HID), _f32),
            pltpu.VMEM((64, HID), _f32),
            pltpu.SemaphoreType.DMA,
            pltpu.SemaphoreType.DMA,
            pltpu.SemaphoreType.DMA,
        ],
        name="sc_scatter",
    )
    def k(m_h, wx_h, dsts_h, oh, ox, acc, idx, b0, b1, zb, s0, s1, r0s):
        cid = lax.axis_index("c")
        sid = lax.axis_index("s")
        r0 = sid * STRIPE
        z16 = jnp.zeros((16,), _f32)

        @pl.loop(0, 64 * HID // 16)
        def zf(kk):
            zb[kk // (HID // 16), pl.ds((kk % (HID // 16)) * 16, 16)] = z16

        @pl.loop(0, STRIPE // 64)
        def zs(kk):
            pltpu.sync_copy(zb, acc.at[pl.ds(r0 + kk * 64, 64)])

        pltpu.sync_copy(dsts_h.at[sid], idx)
        plsc.subcore_barrier()

        def scatter_loop(src_h):
            def load(j, buf, sem):
                e0 = (sid * CPTS + j) * CH
                pltpu.async_copy(src_h.at[pl.ds(e0, CH)], buf, sem)

            def lwait(buf, sem):
                pltpu.make_async_copy(src_h.at[pl.ds(0, CH)], buf, sem).wait()

            load(0, b0, s0)
            load(1, b1, s1)

            @pl.loop(0, CPTS, step=2)
            def body(j0):
                lwait(b0, s0)
                pltpu.sync_copy(b0, acc.at[idx.at[j0]], add=True)

                @pl.when(j0 + 2 < CPTS)
                def _():
                    load(j0 + 2, b0, s0)

                lwait(b1, s1)
                pltpu.sync_copy(b1, acc.at[idx.at[j0 + 1]], add=True)

                @pl.when(j0 + 3 < CPTS)
                def _():
                    load(j0 + 3, b1, s1)

        @pl.when(cid == 0)
        def _():
            scatter_loop(m_h)

        @pl.when(cid == 1)
        def _():
            scatter_loop(wx_h)

        plsc.subcore_barrier()

        def readout(out_h):
            @pl.loop(0, STRIPE // 64)
            def ro(kk):
                pltpu.sync_copy(acc.at[pl.ds(r0 + kk * 64, 64)], zb)
                pltpu.sync_copy(zb, out_h.at[pl.ds(r0 + kk * 64, 64)])

        @pl.when(cid == 0)
        def _():
            readout(oh)

        @pl.when(cid == 1)
        def _():
            readout(ox)

    return k(Mv, WX, dsts)


def _full(shape):
    return pl.BlockSpec(shape, lambda i: (0,) * len(shape))


def _rows(bshape):
    return pl.BlockSpec(bshape, lambda i: (i,) + (0,) * (len(bshape) - 1))


def _axbx(a, b, x):
    xp = jnp.concatenate([x, jnp.zeros((x.shape[0], 128 - XW), _f32)], axis=1)
    return (jnp.concatenate([a, xp], axis=1),
            jnp.concatenate([b, xp], axis=1))


def _tc_init(S, Xp, emb_p, we1a0, we1b0):
    def body(s_ref, x_ref, emb_ref, wa_ref, wb_ref, h_ref, ax_ref, bx_ref):
        s = s_ref[...]
        oh = (s == lax.broadcasted_iota(jnp.int32, (1, 32), 1)).astype(_f32)
        h = _mm(oh, emb_ref[...])
        h_ref[...] = h
        ax, bx = _axbx(_mm(h, wa_ref[...]), _mm(h, wb_ref[...]), x_ref[...])
        ax_ref[...] = ax
        bx_ref[...] = bx

    return pl.pallas_call(
        body,
        grid=(N // NTC,),
        in_specs=[_rows((NTC, 1)), _rows((NTC, XW)), _full((32, HID)),
                  _full((HID, HID)), _full((HID, HID))],
        out_specs=[_rows((NTC, HID)), _rows((NTC, AXW)), _rows((NTC, AXW))],
        out_shape=[jax.ShapeDtypeStruct((N, HID), _f32),
                   jax.ShapeDtypeStruct((N, AXW), _f32),
                   jax.ShapeDtypeStruct((N, AXW), _f32)],
    )(S, Xp, emb_p, we1a0, we1b0)


def _tc_edge(GA, GB, ru, rv, wsym, we1r, we2, wxp, re):
    def body(ga_ref, gb_ref, ru_ref, rv_ref, wsym_ref,
             we1r_ref, we2_ref, wx_ref, re_ref, m_ref, wx_out_ref):
        ga = ga_ref[...]
        gb = gb_ref[...]
        xd = ga[:, HID:HID + XW] - gb[:, HID:HID + XW]
        p = _mm(xd, ru_ref[...])
        q = _mm(xd, rv_ref[...])
        rad = (p[:, :PW] * q[:, :PW]
               + p[:, PW:2 * PW] * q[:, PW:2 * PW]
               + p[:, 2 * PW:] * q[:, 2 * PW:])
        radn = rad / (1.0 + jnp.abs(rad))
        r = _silu(_mm(radn, wsym_ref[...]))
        m1 = _silu(ga[:, :HID] + gb[:, :HID] + _mm(r, we1r_ref[...]))
        m = _silu(_mm(m1, we2_ref[...]))
        cw = jnp.tanh(_mm(m, wx_ref[...]))
        m_ref[...] = m
        wxd = xd * _mm(cw, re_ref[...])
        wx_out_ref[...] = jnp.concatenate(
            [wxd, jnp.zeros((ET, 112 - XW), _f32), jnp.ones((ET, 16), _f32)],
            axis=1)

    return pl.pallas_call(
        body,
        grid=(EP // ET,),
        in_specs=[_rows((ET, AXW)), _rows((ET, AXW)),
                  _full((XW, 3 * PW)), _full((XW, 3 * PW)),
                  _full((PW, HID)), _full((HID, HID)), _full((HID, HID)),
                  _full((HID, 16)), _full((16, XW))],
        out_specs=[_rows((ET, HID)), _rows((ET, HID))],
        out_shape=[jax.ShapeDtypeStruct((EP, HID), _f32),
                   jax.ShapeDtypeStruct((EP, HID), _f32)],
    )(GA, GB, ru, rv, wsym, we1r, we2, wxp, re)


def _node_common(h_ref, hagg_ref, wh1a_ref, wh1b_ref, wh2_ref):
    h = h_ref[...]
    t = _silu(_mm(h, wh1a_ref[...]) + _mm(hagg_ref[...], wh1b_ref[...]))
    return h + _mm(t, wh2_ref[...])


def _x_common(x_ref, xq_ref):
    xq = xq_ref[...]
    deg = xq[:, 112:113]
    return x_ref[...] + xq[:, :XW] / (deg + 1.0)


def _tc_node_mid(h, hp, x, xq, wh1a, wh1b, wh2, we1an, we1bn):
    def body(h_ref, hp_ref, x_ref, xq_ref,
             wh1a_ref, wh1b_ref, wh2_ref, wan_ref, wbn_ref,
             ho_ref, xo_ref, ax_ref, bx_ref):
        hn = _node_common(h_ref, hp_ref, wh1a_ref, wh1b_ref, wh2_ref)
        xn = _x_common(x_ref, xq_ref)
        ho_ref[...] = hn
        xo_ref[...] = xn
        ax, bx = _axbx(_mm(hn, wan_ref[...]), _mm(hn, wbn_ref[...]), xn)
        ax_ref[...] = ax
        bx_ref[...] = bx

    return pl.pallas_call(
        body,
        grid=(N // NTC,),
        in_specs=[_rows((NTC, HID))] * 2 + [_rows((NTC, XW))]
                 + [_rows((NTC, HID))] + [_full((HID, HID))] * 5,
        out_specs=[_rows((NTC, HID)), _rows((NTC, XW)),
                   _rows((NTC, AXW)), _rows((NTC, AXW))],
        out_shape=[jax.ShapeDtypeStruct((N, HID), _f32),
                   jax.ShapeDtypeStruct((N, XW), _f32),
                   jax.ShapeDtypeStruct((N, AXW), _f32),
                   jax.ShapeDtypeStruct((N, AXW), _f32)],
    )(h, hp, x, xq, wh1a, wh1b, wh2, we1an, we1bn)


def _tc_node_round(h, hp, x, xq, S, emb_p,
                   wh1a, wh1b, wh2, wm1, wm2, we1a0, we1b0):
    def body(h_ref, hp_ref, x_ref, xq_ref, s_ref, emb_ref,
             wh1a_ref, wh1b_ref, wh2_ref, wm1_ref, wm2_ref,
             wa_ref, wb_ref, ho_ref, xo_ref, ax_ref, bx_ref):
        hn = _node_common(h_ref, hp_ref, wh1a_ref, wh1b_ref, wh2_ref)
        xn = _x_common(x_ref, xq_ref)
        xo_ref[...] = xn
        mem = _mm(_silu(_mm(_silu(hn), wm1_ref[...])), wm2_ref[...])
        oh = (s_ref[...] == lax.broadcasted_iota(jnp.int32, (1, 32), 1)
              ).astype(_f32)
        hnew = _mm(oh, emb_ref[...]) + mem
        ho_ref[...] = hnew
        ax, bx = _axbx(_mm(hnew, wa_ref[...]), _mm(hnew, wb_ref[...]), xn)
        ax_ref[...] = ax
        bx_ref[...] = bx

    return pl.pallas_call(
        body,
        grid=(N // NTC,),
        in_specs=[_rows((NTC, HID))] * 2 + [_rows((NTC, XW))]
                 + [_rows((NTC, HID))] + [_rows((NTC, 1))]
                 + [_full((32, HID))] + [_full((HID, HID))] * 7,
        out_specs=[_rows((NTC, HID)), _rows((NTC, XW)),
                   _rows((NTC, AXW)), _rows((NTC, AXW))],
        out_shape=[jax.ShapeDtypeStruct((N, HID), _f32),
                   jax.ShapeDtypeStruct((N, XW), _f32),
                   jax.ShapeDtypeStruct((N, AXW), _f32),
                   jax.ShapeDtypeStruct((N, AXW), _f32)],
    )(h, hp, x, xq, S, emb_p, wh1a, wh1b, wh2, wm1, wm2, we1a0, we1b0)


def _tc_node_final(h, hp, wh1a, wh1b, wh2, wr1, wr2):
    def body(h_ref, hp_ref, wh1a_ref, wh1b_ref, wh2_ref,
             wr1_ref, wr2_ref, o_ref):
        hn = _node_common(h_ref, hp_ref, wh1a_ref, wh1b_ref, wh2_ref)
        o_ref[...] = _mm(_silu(_mm(_silu(hn), wr1_ref[...])), wr2_ref[...])

    return pl.pallas_call(
        body,
        grid=(N // NTC,),
        in_specs=[_rows((NTC, HID))] * 2 + [_full((HID, HID))] * 4
                 + [_full((HID, NCLS))],
        out_specs=_rows((NTC, NCLS)),
        out_shape=jax.ShapeDtypeStruct((N, NCLS), _f32),
    )(h, hp, wh1a, wh1b, wh2, wr1, wr2)


def kernel(X, S, edge_index, emb, W_rad, W_e1, W_e2, W_x, W_h1, W_h2,
           W_m1, W_m2, W_r1, W_r2):
    ru = jnp.asarray(_RU_NP)
    rv = jnp.asarray(_RV_NP)
    re = jnp.asarray(_RE_NP)
    icd = jnp.asarray(_ICD_NP)
    idc = jnp.asarray(_IDC_NP)
    offd = jnp.asarray(_OFFD_NP)

    Xp = jnp.pad(X.reshape(N, C * 3), ((0, 0), (0, XW - C * 3)))
    emb_p = jnp.pad(emb, ((0, 32 - NCLS), (0, 0)))
    S32 = S.astype(jnp.int32).reshape(N, 1)

    src = edge_index[0].astype(jnp.int32)
    dst = edge_index[1].astype(jnp.int32)
    padn = EP - E
    srcg = jnp.concatenate([src, jnp.zeros((padn,), jnp.int32)]
                           ).reshape(32, CPTG, CH)
    dstg = jnp.concatenate([dst, jnp.zeros((padn,), jnp.int32)]
                           ).reshape(32, CPTG, CH)
    dsts = jnp.concatenate([dst, jnp.full((padn,), N, jnp.int32)]
                           ).reshape(16, CPTS, CH)

    wsym = [jnp.pad(W_rad[l][icd] + W_rad[l][idc] * offd[:, None],
                    ((0, PW - NPAIR), (0, 0))) for l in range(NL)]
    we1a = [W_e1[l, :HID] for l in range(NL)]
    we1b = [W_e1[l, HID:2 * HID] for l in range(NL)]
    we1r = [W_e1[l, 2 * HID:] for l in range(NL)]
    wxp = [jnp.pad(W_x[l], ((0, 0), (0, 16 - C))) for l in range(NL)]
    wh1a = [W_h1[l, :HID] for l in range(NL)]
    wh1b = [W_h1[l, HID:] for l in range(NL)]

    h, AX, BX = _tc_init(S32, Xp, emb_p, we1a[0], we1b[0])
    x = Xp
    logits = None
    for r in range(ROUNDS):
        for l in range(NL):
            GA, GB = _sc_gather(AX, BX, srcg, dstg)
            Mv, WX = _tc_edge(GA, GB, ru, rv, wsym[l],
                              we1r[l], W_e2[l], wxp[l], re)
            Hp, Xq = _sc_scatter(Mv, WX, dsts)
            hp = Hp[:N]
            xq = Xq[:N]
            last = l == NL - 1
            if not last:
                h, x, AX, BX = _tc_node_mid(h, hp, x, xq,
                                            wh1a[l], wh1b[l], W_h2[l],
                                            we1a[l + 1], we1b[l + 1])
            elif r < ROUNDS - 1:
                h, x, AX, BX = _tc_node_round(h, hp, x, xq, S32, emb_p,
                                              wh1a[l], wh1b[l], W_h2[l],
                                              W_m1, W_m2, we1a[0], we1b[0])
            else:
                logits = _tc_node_final(h, hp, wh1a[l], wh1b[l],
                                        W_h2[l], W_r1, W_r2)
    return logits


# speedup vs baseline: 1.5403x; 1.5403x over previous
import functools

import numpy as np
import jax
import jax.numpy as jnp
from jax import lax
from jax.experimental import pallas as pl
from jax.experimental.pallas import tpu as pltpu
from jax.experimental.pallas import tpu_sc as plsc

N = 10000
E = 90000
C = 14
HID = 128
NCLS = 25
NL = 3
ROUNDS = 3

XW = 48
NPAIR = C * (C + 1) // 2
PW = 128
AXW = HID + 128

CH = 112
CPTG = 26
CPTS = 52
EP = 32 * CH * CPTG
NT = 10240
STRIPE = NT // 16

ET = 1024
NTC = 2000

_f32 = jnp.float32


def _mm(a, b):
    return lax.dot_general(a, b, (((a.ndim - 1,), (0,)), ((), ())),
                           preferred_element_type=_f32)


def _silu(x):
    y = 0.5 * x
    return y + y * jnp.tanh(y)


def _np_expand_consts():
    pairs = [(c, d) for c in range(C) for d in range(c, C)]
    ru = np.zeros((XW, 3 * PW), np.float32)
    rv = np.zeros((XW, 3 * PW), np.float32)
    for p, (c, d) in enumerate(pairs):
        for i in range(3):
            ru[c * 3 + i, i * PW + p] = 1.0
            rv[d * 3 + i, i * PW + p] = 1.0
    re = np.zeros((16, XW), np.float32)
    for c in range(C):
        for i in range(3):
            re[c, c * 3 + i] = 1.0
    idx_cd = np.array([c * C + d for (c, d) in pairs], np.int32)
    idx_dc = np.array([d * C + c for (c, d) in pairs], np.int32)
    offd = np.array([1.0 if c != d else 0.0 for (c, d) in pairs], np.float32)
    return ru, rv, re, idx_cd, idx_dc, offd


_RU_NP, _RV_NP, _RE_NP, _ICD_NP, _IDC_NP, _OFFD_NP = _np_expand_consts()


@functools.cache
def _sc_mesh():
    return plsc.VectorSubcoreMesh(core_axis_name="c", subcore_axis_name="s")


def _sc_gather(AX, BX, srcg, dstg):
    @functools.partial(
        pl.kernel,
        out_type=[
            jax.ShapeDtypeStruct((EP, AXW), _f32),
            jax.ShapeDtypeStruct((EP, AXW), _f32),
        ],
        mesh=_sc_mesh(),
        scratch_types=[
            pltpu.VMEM((CPTG, CH), jnp.int32),
            pltpu.VMEM((CPTG, CH), jnp.int32),
            pltpu.VMEM((CH, AXW), _f32),
            pltpu.VMEM((CH, AXW), _f32),
            pltpu.VMEM((CH, AXW), _f32),
            pltpu.VMEM((CH, AXW), _f32),
            pltpu.SemaphoreType.DMA,
            pltpu.SemaphoreType.DMA,
            pltpu.SemaphoreType.DMA,
            pltpu.SemaphoreType.DMA,
        ],
        name="sc_gather",
    )
    def k(ax_h, bx_h, srcg_h, dstg_h, oa, ob,
          isrc, idst, ba0, bb0, ba1, bb1, gs0, gs1, ws0, ws1):
        w = lax.axis_index("c") * 16 + lax.axis_index("s")
        pltpu.sync_copy(srcg_h.at[w], isrc)
        pltpu.sync_copy(dstg_h.at[w], idst)

        @pl.loop(0, CPTG, step=2)
        def body(j0):
            g0a = pltpu.async_copy(ax_h.at[isrc.at[j0]], ba0, gs0)
            g0b = pltpu.async_copy(bx_h.at[idst.at[j0]], bb0, gs0)
            g1a = pltpu.async_copy(ax_h.at[isrc.at[j0 + 1]], ba1, gs1)
            g1b = pltpu.async_copy(bx_h.at[idst.at[j0 + 1]], bb1, gs1)
            e0 = (w * CPTG + j0) * CH
            g0a.wait(); g0b.wait()
            w0a = pltpu.async_copy(ba0, oa.at[pl.ds(e0, CH)], ws0)
            w0b = pltpu.async_copy(bb0, ob.at[pl.ds(e0, CH)], ws0)
            g1a.wait(); g1b.wait()
            w1a = pltpu.async_copy(ba1, oa.at[pl.ds(e0 + CH, CH)], ws1)
            w1b = pltpu.async_copy(bb1, ob.at[pl.ds(e0 + CH, CH)], ws1)
            w0a.wait(); w0b.wait(); w1a.wait(); w1b.wait()

    return k(AX, BX, srcg, dstg)


def _sc_scatter(Mv, WX, dsts):
    @functools.partial(
        pl.kernel,
        out_type=[
            jax.ShapeDtypeStruct((NT, HID), _f32),
            jax.ShapeDtypeStruct((NT, HID), _f32),
        ],
        mesh=_sc_mesh(),
        scratch_types=[
            pltpu.VMEM_SHARED((NT, HID), _f32),
            pltpu.VMEM((CPTS, CH), jnp.int32),
            pltpu.VMEM((CH, HID), _f32),
            pltpu.VMEM((CH, HID), _f32),
            pltpu.VMEM((64, HID), _f32),
            pltpu.SemaphoreType.DMA,
            pltpu.SemaphoreType.DMA,
            pltpu.SemaphoreType.DMA,
        ],
        name="sc_scatter",
    )
    def k(m_h, wx_h, dsts_h, oh, ox, acc, idx, b0, b1, zb, s0, s1, r0s):
        cid = lax.axis_index("c")
        sid = lax.axis_index("s")
        r0 = sid * STRIPE
        z16 = jnp.zeros((16,), _f32)

        @pl.loop(0, 64 * HID // 16)
        def zf(kk):
            zb[kk // (HID // 16), pl.ds((kk % (HID // 16)) * 16, 16)] = z16

        @pl.loop(0, STRIPE // 64)
        def zs(kk):
            pltpu.sync_copy(zb, acc.at[pl.ds(r0 + kk * 64, 64)])

        pltpu.sync_copy(dsts_h.at[sid], idx)
        plsc.subcore_barrier()

        def scatter_loop(src_h):
            def load(j, buf, sem):
                e0 = (sid * CPTS + j) * CH
                pltpu.async_copy(src_h.at[pl.ds(e0, CH)], buf, sem)

            def lwait(buf, sem):
                pltpu.make_async_copy(src_h.at[pl.ds(0, CH)], buf, sem).wait()

            load(0, b0, s0)
            load(1, b1, s1)

            @pl.loop(0, CPTS, step=2)
            def body(j0):
                lwait(b0, s0)
                pltpu.sync_copy(b0, acc.at[idx.at[j0]], add=True)

                @pl.when(j0 + 2 < CPTS)
                def _():
                    load(j0 + 2, b0, s0)

                lwait(b1, s1)
                pltpu.sync_copy(b1, acc.at[idx.at[j0 + 1]], add=True)

                @pl.when(j0 + 3 < CPTS)
                def _():
                    load(j0 + 3, b1, s1)

        @pl.when(cid == 0)
        def _():
            scatter_loop(m_h)

        @pl.when(cid == 1)
        def _():
            scatter_loop(wx_h)

        plsc.subcore_barrier()

        def readout(out_h):
            @pl.loop(0, STRIPE // 64)
            def ro(kk):
                pltpu.sync_copy(acc.at[pl.ds(r0 + kk * 64, 64)], zb)
                pltpu.sync_copy(zb, out_h.at[pl.ds(r0 + kk * 64, 64)])

        @pl.when(cid == 0)
        def _():
            readout(oh)

        @pl.when(cid == 1)
        def _():
            readout(ox)

    return k(Mv, WX, dsts)


def _full(shape):
    return pl.BlockSpec(shape, lambda i: (0,) * len(shape))


def _rows(bshape):
    return pl.BlockSpec(bshape, lambda i: (i,) + (0,) * (len(bshape) - 1))


def _axbx(a, b, x):
    xp = jnp.concatenate([x, jnp.zeros((x.shape[0], 128 - XW), _f32)], axis=1)
    return (jnp.concatenate([a, xp], axis=1),
            jnp.concatenate([b, xp], axis=1))


def _tc_init(S, Xp, emb_p, we1a0, we1b0):
    def body(s_ref, x_ref, emb_ref, wa_ref, wb_ref, h_ref, ax_ref, bx_ref):
        s = s_ref[...]
        oh = (s == lax.broadcasted_iota(jnp.int32, (1, 32), 1)).astype(_f32)
        h = _mm(oh, emb_ref[...])
        h_ref[...] = h
        ax, bx = _axbx(_mm(h, wa_ref[...]), _mm(h, wb_ref[...]), x_ref[...])
        ax_ref[...] = ax
        bx_ref[...] = bx

    return pl.pallas_call(
        body,
        grid=(N // NTC,),
        in_specs=[_rows((NTC, 1)), _rows((NTC, XW)), _full((32, HID)),
                  _full((HID, HID)), _full((HID, HID))],
        out_specs=[_rows((NTC, HID)), _rows((NTC, AXW)), _rows((NTC, AXW))],
        out_shape=[jax.ShapeDtypeStruct((N, HID), _f32),
                   jax.ShapeDtypeStruct((N, AXW), _f32),
                   jax.ShapeDtypeStruct((N, AXW), _f32)],
    )(S, Xp, emb_p, we1a0, we1b0)


def _tc_edge(GA, GB, ru, rv, wsym, we1r, we2, wxp, re):
    def body(ga_ref, gb_ref, ru_ref, rv_ref, wsym_ref,
             we1r_ref, we2_ref, wx_ref, re_ref, m_ref, wx_out_ref):
        ga = ga_ref[...]
        gb = gb_ref[...]
        xd = ga[:, HID:HID + XW] - gb[:, HID:HID + XW]
        p = _mm(xd, ru_ref[...])
        q = _mm(xd, rv_ref[...])
        rad = (p[:, :PW] * q[:, :PW]
               + p[:, PW:2 * PW] * q[:, PW:2 * PW]
               + p[:, 2 * PW:] * q[:, 2 * PW:])
        radn = rad / (1.0 + jnp.abs(rad))
        r = _silu(_mm(radn, wsym_ref[...]))
        m1 = _silu(ga[:, :HID] + gb[:, :HID] + _mm(r, we1r_ref[...]))
        m = _silu(_mm(m1, we2_ref[...]))
        cw = jnp.tanh(_mm(m, wx_ref[...]))
        m_ref[...] = m
        wxd = xd * _mm(cw, re_ref[...])
        wx_out_ref[...] = jnp.concatenate(
            [wxd, jnp.zeros((ET, 112 - XW), _f32), jnp.ones((ET, 16), _f32)],
            axis=1)

    return pl.pallas_call(
        body,
        grid=(EP // ET,),
        in_specs=[_rows((ET, AXW)), _rows((ET, AXW)),
                  _full((XW, 3 * PW)), _full((XW, 3 * PW)),
                  _full((PW, HID)), _full((HID, HID)), _full((HID, HID)),
                  _full((HID, 16)), _full((16, XW))],
        out_specs=[_rows((ET, HID)), _rows((ET, HID))],
        out_shape=[jax.ShapeDtypeStruct((EP, HID), _f32),
                   jax.ShapeDtypeStruct((EP, HID), _f32)],
    )(GA, GB, ru, rv, wsym, we1r, we2, wxp, re)


def _node_common(h_ref, hagg_ref, wh1a_ref, wh1b_ref, wh2_ref):
    h = h_ref[...]
    t = _silu(_mm(h, wh1a_ref[...]) + _mm(hagg_ref[...], wh1b_ref[...]))
    return h + _mm(t, wh2_ref[...])


def _x_common(x_ref, xq_ref):
    xq = xq_ref[...]
    deg = xq[:, 112:113]
    return x_ref[...] + xq[:, :XW] / (deg + 1.0)


def _tc_node_mid(h, hp, x, xq, wh1a, wh1b, wh2, we1an, we1bn):
    def body(h_ref, hp_ref, x_ref, xq_ref,
             wh1a_ref, wh1b_ref, wh2_ref, wan_ref, wbn_ref,
             ho_ref, xo_ref, ax_ref, bx_ref):
        hn = _node_common(h_ref, hp_ref, wh1a_ref, wh1b_ref, wh2_ref)
        xn = _x_common(x_ref, xq_ref)
        ho_ref[...] = hn
        xo_ref[...] = xn
        ax, bx = _axbx(_mm(hn, wan_ref[...]), _mm(hn, wbn_ref[...]), xn)
        ax_ref[...] = ax
        bx_ref[...] = bx

    return pl.pallas_call(
        body,
        grid=(N // NTC,),
        in_specs=[_rows((NTC, HID))] * 2 + [_rows((NTC, XW))]
                 + [_rows((NTC, HID))] + [_full((HID, HID))] * 5,
        out_specs=[_rows((NTC, HID)), _rows((NTC, XW)),
                   _rows((NTC, AXW)), _rows((NTC, AXW))],
        out_shape=[jax.ShapeDtypeStruct((N, HID), _f32),
                   jax.ShapeDtypeStruct((N, XW), _f32),
                   jax.ShapeDtypeStruct((N, AXW), _f32),
                   jax.ShapeDtypeStruct((N, AXW), _f32)],
    )(h, hp, x, xq, wh1a, wh1b, wh2, we1an, we1bn)


def _tc_node_round(h, hp, x, xq, S, emb_p,
                   wh1a, wh1b, wh2, wm1, wm2, we1a0, we1b0):
    def body(h_ref, hp_ref, x_ref, xq_ref, s_ref, emb_ref,
             wh1a_ref, wh1b_ref, wh2_ref, wm1_ref, wm2_ref,
             wa_ref, wb_ref, ho_ref, xo_ref, ax_ref, bx_ref):
        hn = _node_common(h_ref, hp_ref, wh1a_ref, wh1b_ref, wh2_ref)
        xn = _x_common(x_ref, xq_ref)
        xo_ref[...] = xn
        mem = _mm(_silu(_mm(_silu(hn), wm1_ref[...])), wm2_ref[...])
        oh = (s_ref[...] == lax.broadcasted_iota(jnp.int32, (1, 32), 1)
              ).astype(_f32)
        hnew = _mm(oh, emb_ref[...]) + mem
        ho_ref[...] = hnew
        ax, bx = _axbx(_mm(hnew, wa_ref[...]), _mm(hnew, wb_ref[...]), xn)
        ax_ref[...] = ax
        bx_ref[...] = bx

    return pl.pallas_call(
        body,
        grid=(N // NTC,),
        in_specs=[_rows((NTC, HID))] * 2 + [_rows((NTC, XW))]
                 + [_rows((NTC, HID))] + [_rows((NTC, 1))]
                 + [_full((32, HID))] + [_full((HID, HID))] * 7,
        out_specs=[_rows((NTC, HID)), _rows((NTC, XW)),
                   _rows((NTC, AXW)), _rows((NTC, AXW))],
        out_shape=[jax.ShapeDtypeStruct((N, HID), _f32),
                   jax.ShapeDtypeStruct((N, XW), _f32),
                   jax.ShapeDtypeStruct((N, AXW), _f32),
                   jax.ShapeDtypeStruct((N, AXW), _f32)],
    )(h, hp, x, xq, S, emb_p, wh1a, wh1b, wh2, wm1, wm2, we1a0, we1b0)


def _tc_node_final(h, hp, wh1a, wh1b, wh2, wr1, wr2):
    def body(h_ref, hp_ref, wh1a_ref, wh1b_ref, wh2_ref,
             wr1_ref, wr2_ref, o_ref):
        hn = _node_common(h_ref, hp_ref, wh1a_ref, wh1b_ref, wh2_ref)
        o_ref[...] = _mm(_silu(_mm(_silu(hn), wr1_ref[...])), wr2_ref[...])

    return pl.pallas_call(
        body,
        grid=(N // NTC,),
        in_specs=[_rows((NTC, HID))] * 2 + [_full((HID, HID))] * 4
                 + [_full((HID, NCLS))],
        out_specs=_rows((NTC, NCLS)),
        out_shape=jax.ShapeDtypeStruct((N, NCLS), _f32),
    )(h, hp, wh1a, wh1b, wh2, wr1, wr2)


def kernel(X, S, edge_index, emb, W_rad, W_e1, W_e2, W_x, W_h1, W_h2,
           W_m1, W_m2, W_r1, W_r2):
    ru = jnp.asarray(_RU_NP)
    rv = jnp.asarray(_RV_NP)
    re = jnp.asarray(_RE_NP)
    icd = jnp.asarray(_ICD_NP)
    idc = jnp.asarray(_IDC_NP)
    offd = jnp.asarray(_OFFD_NP)

    Xp = jnp.pad(X.reshape(N, C * 3), ((0, 0), (0, XW - C * 3)))
    emb_p = jnp.pad(emb, ((0, 32 - NCLS), (0, 0)))
    S32 = S.astype(jnp.int32).reshape(N, 1)

    src = edge_index[0].astype(jnp.int32)
    dst = edge_index[1].astype(jnp.int32)
    padn = EP - E
    spread = jnp.arange(padn, dtype=jnp.int32)
    srcg = jnp.concatenate([src, spread % N]).reshape(32, CPTG, CH)
    dstg = jnp.concatenate([dst, spread % N]).reshape(32, CPTG, CH)
    dsts = jnp.concatenate([dst, N + spread % (NT - N)]
                           ).reshape(16, CPTS, CH)

    wsym = [jnp.pad(W_rad[l][icd] + W_rad[l][idc] * offd[:, None],
                    ((0, PW - NPAIR), (0, 0))) for l in range(NL)]
    we1a = [W_e1[l, :HID] for l in range(NL)]
    we1b = [W_e1[l, HID:2 * HID] for l in range(NL)]
    we1r = [W_e1[l, 2 * HID:] for l in range(NL)]
    wxp = [jnp.pad(W_x[l], ((0, 0), (0, 16 - C))) for l in range(NL)]
    wh1a = [W_h1[l, :HID] for l in range(NL)]
    wh1b = [W_h1[l, HID:] for l in range(NL)]

    h, AX, BX = _tc_init(S32, Xp, emb_p, we1a[0], we1b[0])
    x = Xp
    logits = None
    for r in range(ROUNDS):
        for l in range(NL):
            GA, GB = _sc_gather(AX, BX, srcg, dstg)
            Mv, WX = _tc_edge(GA, GB, ru, rv, wsym[l],
                              we1r[l], W_e2[l], wxp[l], re)
            Hp, Xq = _sc_scatter(Mv, WX, dsts)
            hp = Hp[:N]
            xq = Xq[:N]
            last = l == NL - 1
            if not last:
                h, x, AX, BX = _tc_node_mid(h, hp, x, xq,
                                            wh1a[l], wh1b[l], W_h2[l],
                                            we1a[l + 1], we1b[l + 1])
            elif r < ROUNDS - 1:
                h, x, AX, BX = _tc_node_round(h, hp, x, xq, S32, emb_p,
                                              wh1a[l], wh1b[l], W_h2[l],
                                              W_m1, W_m2, we1a[0], we1b[0])
            else:
                logits = _tc_node_final(h, hp, wh1a[l], wh1b[l],
                                        W_h2[l], W_r1, W_r2)
    return logits


# speedup vs baseline: 1.9451x; 1.2628x over previous
import functools

import numpy as np
import jax
import jax.numpy as jnp
from jax import lax
from jax.experimental import pallas as pl
from jax.experimental.pallas import tpu as pltpu
from jax.experimental.pallas import tpu_sc as plsc

N = 10000
E = 90000
C = 14
HID = 128
NCLS = 25
NL = 3
ROUNDS = 3

XW = 48
NPAIR = C * (C + 1) // 2
PW = 128
PKW = 128

CH = 112
CPTG = 14
CPTS = 56
EPH = 32 * CH * CPTG
EP = 2 * EPH
NT = 10240
STRIPE = NT // 16

ET = 1024
NTC = 2000

_f32 = jnp.float32
_bf16 = jnp.bfloat16


def _mm(a, b):
    return lax.dot_general(a, b, (((a.ndim - 1,), (0,)), ((), ())),
                           preferred_element_type=_f32)


def _mmb(a, b):
    return lax.dot_general(a.astype(_bf16), b.astype(_bf16),
                           (((a.ndim - 1,), (0,)), ((), ())),
                           preferred_element_type=_f32)


def _silu(x):
    y = 0.5 * x
    return y + y * jnp.tanh(y)


def _np_expand_consts():
    pairs = [(c, d) for c in range(C) for d in range(c, C)]
    ru = np.zeros((XW, 3 * PW), np.float32)
    rv = np.zeros((XW, 3 * PW), np.float32)
    for p, (c, d) in enumerate(pairs):
        for i in range(3):
            ru[c * 3 + i, i * PW + p] = 1.0
            rv[d * 3 + i, i * PW + p] = 1.0
    re = np.zeros((16, XW), np.float32)
    for c in range(C):
        for i in range(3):
            re[c, c * 3 + i] = 1.0
    idx_cd = np.array([c * C + d for (c, d) in pairs], np.int32)
    idx_dc = np.array([d * C + c for (c, d) in pairs], np.int32)
    offd = np.array([1.0 if c != d else 0.0 for (c, d) in pairs], np.float32)
    return ru, rv, re, idx_cd, idx_dc, offd


_RU_NP, _RV_NP, _RE_NP, _ICD_NP, _IDC_NP, _OFFD_NP = _np_expand_consts()


@functools.cache
def _sc_mesh():
    return plsc.VectorSubcoreMesh(core_axis_name="c", subcore_axis_name="s")


def _sc_gather(AX, BX, srcg, dstg):
    @functools.partial(
        pl.kernel,
        out_type=[
            jax.ShapeDtypeStruct((EPH, PKW), jnp.int32),
            jax.ShapeDtypeStruct((EPH, PKW), jnp.int32),
        ],
        mesh=_sc_mesh(),
        scratch_types=[
            pltpu.VMEM((CPTG, CH), jnp.int32),
            pltpu.VMEM((CPTG, CH), jnp.int32),
            pltpu.VMEM((CH, PKW), jnp.int32),
            pltpu.VMEM((CH, PKW), jnp.int32),
            pltpu.VMEM((CH, PKW), jnp.int32),
            pltpu.VMEM((CH, PKW), jnp.int32),
            pltpu.SemaphoreType.DMA,
            pltpu.SemaphoreType.DMA,
            pltpu.SemaphoreType.DMA,
            pltpu.SemaphoreType.DMA,
        ],
        name="sc_gather",
    )
    def k(ax_h, bx_h, srcg_h, dstg_h, oa, ob,
          isrc, idst, ba0, bb0, ba1, bb1, gs0, gs1, ws0, ws1):
        w = lax.axis_index("c") * 16 + lax.axis_index("s")
        pltpu.sync_copy(srcg_h.at[w], isrc)
        pltpu.sync_copy(dstg_h.at[w], idst)

        @pl.loop(0, CPTG, step=2)
        def body(j0):
            g0a = pltpu.async_copy(ax_h.at[isrc.at[j0]], ba0, gs0)
            g0b = pltpu.async_copy(bx_h.at[idst.at[j0]], bb0, gs0)
            g1a = pltpu.async_copy(ax_h.at[isrc.at[j0 + 1]], ba1, gs1)
            g1b = pltpu.async_copy(bx_h.at[idst.at[j0 + 1]], bb1, gs1)
            e0 = (w * CPTG + j0) * CH
            g0a.wait(); g0b.wait()
            w0a = pltpu.async_copy(ba0, oa.at[pl.ds(e0, CH)], ws0)
            w0b = pltpu.async_copy(bb0, ob.at[pl.ds(e0, CH)], ws0)
            g1a.wait(); g1b.wait()
            w1a = pltpu.async_copy(ba1, oa.at[pl.ds(e0 + CH, CH)], ws1)
            w1b = pltpu.async_copy(bb1, ob.at[pl.ds(e0 + CH, CH)], ws1)
            w0a.wait(); w0b.wait(); w1a.wait(); w1b.wait()

    return k(AX, BX, srcg, dstg)


def _sc_scatter(Mv1, WX1, Mv2, WX2, dsts):
    @functools.partial(
        pl.kernel,
        out_type=[
            jax.ShapeDtypeStruct((NT, HID), _f32),
            jax.ShapeDtypeStruct((NT, HID), _f32),
        ],
        mesh=_sc_mesh(),
        scratch_types=[
            pltpu.VMEM_SHARED((NT, HID), _f32),
            pltpu.VMEM((CPTS, CH), jnp.int32),
            pltpu.VMEM((CH, HID), _f32),
            pltpu.VMEM((CH, HID), _f32),
            pltpu.VMEM((64, HID), _f32),
            pltpu.SemaphoreType.DMA,
            pltpu.SemaphoreType.DMA,
            pltpu.SemaphoreType.DMA,
        ],
        name="sc_scatter",
    )
    def k(m1_h, wx1_h, m2_h, wx2_h, dsts_h, oh, ox,
          acc, idx, b0, b1, zb, s0, s1, r0s):
        cid = lax.axis_index("c")
        sid = lax.axis_index("s")
        r0 = sid * STRIPE
        z16 = jnp.zeros((16,), _f32)

        @pl.loop(0, 64 * HID // 16)
        def zf(kk):
            zb[kk // (HID // 16), pl.ds((kk % (HID // 16)) * 16, 16)] = z16

        @pl.loop(0, STRIPE // 64)
        def zs(kk):
            pltpu.sync_copy(zb, acc.at[pl.ds(r0 + kk * 64, 64)])

        pltpu.sync_copy(dsts_h.at[sid], idx)
        plsc.subcore_barrier()

        wb = sid % 8

        def scatter_loop(src_h):
            def load(j, buf, sem):
                e0 = (wb * CPTS + j) * CH
                pltpu.async_copy(src_h.at[pl.ds(e0, CH)], buf, sem)

            def lwait(buf, sem):
                pltpu.make_async_copy(src_h.at[pl.ds(0, CH)], buf, sem).wait()

            load(0, b0, s0)
            load(1, b1, s1)

            @pl.loop(0, CPTS, step=2)
            def body(j0):
                lwait(b0, s0)
                pltpu.sync_copy(b0, acc.at[idx.at[j0]], add=True)

                @pl.when(j0 + 2 < CPTS)
                def _():
                    load(j0 + 2, b0, s0)

                lwait(b1, s1)
                pltpu.sync_copy(b1, acc.at[idx.at[j0 + 1]], add=True)

                @pl.when(j0 + 3 < CPTS)
                def _():
                    load(j0 + 3, b1, s1)

        @pl.when(jnp.logical_and(cid == 0, sid < 8))
        def _():
            scatter_loop(m1_h)

        @pl.when(jnp.logical_and(cid == 0, sid >= 8))
        def _():
            scatter_loop(m2_h)

        @pl.when(jnp.logical_and(cid == 1, sid < 8))
        def _():
            scatter_loop(wx1_h)

        @pl.when(jnp.logical_and(cid == 1, sid >= 8))
        def _():
            scatter_loop(wx2_h)

        plsc.subcore_barrier()

        def readout(out_h):
            @pl.loop(0, STRIPE // 64)
            def ro(kk):
                pltpu.sync_copy(acc.at[pl.ds(r0 + kk * 64, 64)], zb)
                pltpu.sync_copy(zb, out_h.at[pl.ds(r0 + kk * 64, 64)])

        @pl.when(cid == 0)
        def _():
            readout(oh)

        @pl.when(cid == 1)
        def _():
            readout(ox)

    return k(Mv1, WX1, Mv2, WX2, dsts)


def _full(shape):
    return pl.BlockSpec(shape, lambda i: (0,) * len(shape))


def _rows(bshape):
    return pl.BlockSpec(bshape, lambda i: (i,) + (0,) * (len(bshape) - 1))


def _pack(a, x):
    xp = jnp.concatenate([x, jnp.zeros((x.shape[0], 128 - XW), _f32)], axis=1)
    ua = lax.bitcast_convert_type(a.astype(_bf16).astype(_f32), jnp.uint32)
    ux = lax.bitcast_convert_type(xp.astype(_bf16).astype(_f32), jnp.uint32)
    word = jnp.bitwise_or(jnp.bitwise_and(ux, jnp.uint32(0xFFFF0000)),
                          lax.shift_right_logical(ua, jnp.uint32(16)))
    return lax.bitcast_convert_type(word, jnp.int32)


def _axbx(a, b, x):
    return _pack(a, x), _pack(b, x)


def _unpack(g_i32):
    u = lax.bitcast_convert_type(g_i32, jnp.uint32)
    a = lax.bitcast_convert_type(lax.shift_left(u, jnp.uint32(16)), _f32)
    x = lax.bitcast_convert_type(
        jnp.bitwise_and(u, jnp.uint32(0xFFFF0000)), _f32)
    return a, x


def _tc_init(S, Xp, emb_p, we1a0, we1b0):
    def body(s_ref, x_ref, emb_ref, wa_ref, wb_ref, h_ref, ax_ref, bx_ref):
        s = s_ref[...]
        oh = (s == lax.broadcasted_iota(jnp.int32, (1, 32), 1)).astype(_f32)
        h = _mm(oh, emb_ref[...])
        h_ref[...] = h
        ax, bx = _axbx(_mm(h, wa_ref[...]), _mm(h, wb_ref[...]), x_ref[...])
        ax_ref[...] = ax
        bx_ref[...] = bx

    return pl.pallas_call(
        body,
        grid=(N // NTC,),
        in_specs=[_rows((NTC, 1)), _rows((NTC, XW)), _full((32, HID)),
                  _full((HID, HID)), _full((HID, HID))],
        out_specs=[_rows((NTC, HID)), _rows((NTC, PKW)), _rows((NTC, PKW))],
        out_shape=[jax.ShapeDtypeStruct((N, HID), _f32),
                   jax.ShapeDtypeStruct((N, PKW), jnp.int32),
                   jax.ShapeDtypeStruct((N, PKW), jnp.int32)],
    )(S, Xp, emb_p, we1a0, we1b0)


def _tc_edge(GA, GB, ru, rv, wsym, we1r, we2, wxp, re):
    def body(ga_ref, gb_ref, ru_ref, rv_ref, wsym_ref,
             we1r_ref, we2_ref, wx_ref, re_ref, m_ref, wx_out_ref):
        a128, xs128 = _unpack(ga_ref[...])
        b128, xdst128 = _unpack(gb_ref[...])
        xd48 = xs128[:, :XW] - xdst128[:, :XW]
        xd = xd48
        p = _mmb(xd48, ru_ref[...])
        q = _mmb(xd48, rv_ref[...])
        rad = (p[:, :PW] * q[:, :PW]
               + p[:, PW:2 * PW] * q[:, PW:2 * PW]
               + p[:, 2 * PW:] * q[:, 2 * PW:])
        radn = rad / (1.0 + jnp.abs(rad))
        r = _silu(_mmb(radn, wsym_ref[...]))
        m1 = _silu(a128 + b128 + _mmb(r, we1r_ref[...]))
        m = _silu(_mmb(m1, we2_ref[...]))
        cw = jnp.tanh(_mmb(m, wx_ref[...]))
        m_ref[...] = m
        wxd = xd * _mmb(cw, re_ref[...])
        wx_out_ref[...] = jnp.concatenate(
            [wxd, jnp.zeros((ET, 112 - XW), _f32), jnp.ones((ET, 16), _f32)],
            axis=1)

    return pl.pallas_call(
        body,
        grid=(EPH // ET,),
        in_specs=[_rows((ET, PKW)), _rows((ET, PKW)),
                  _full((XW, 3 * PW)), _full((XW, 3 * PW)),
                  _full((PW, HID)), _full((HID, HID)), _full((HID, HID)),
                  _full((HID, 16)), _full((16, XW))],
        out_specs=[_rows((ET, HID)), _rows((ET, HID))],
        out_shape=[jax.ShapeDtypeStruct((EPH, HID), _f32),
                   jax.ShapeDtypeStruct((EPH, HID), _f32)],
    )(GA, GB, ru, rv, wsym, we1r, we2, wxp, re)


def _node_common(h_ref, hp_ref, wh1a_ref, wh1b_ref, wh2_ref):
    h = h_ref[...]
    t = _silu(_mm(h, wh1a_ref[...]) + _mm(hp_ref[...], wh1b_ref[...]))
    return h + _mm(t, wh2_ref[...])


def _x_common(x_ref, xq_ref):
    xq = xq_ref[...]
    deg = xq[:, 112:113]
    return x_ref[...] + xq[:, :XW] / (deg + 1.0)


def _tc_node_mid(h, hp, x, xq, wh1a, wh1b, wh2, we1an, we1bn):
    def body(h_ref, hp_ref, x_ref, xq_ref,
             wh1a_ref, wh1b_ref, wh2_ref, wan_ref, wbn_ref,
             ho_ref, xo_ref, ax_ref, bx_ref):
        hn = _node_common(h_ref, hp_ref, wh1a_ref, wh1b_ref, wh2_ref)
        xn = _x_common(x_ref, xq_ref)
        ho_ref[...] = hn
        xo_ref[...] = xn
        ax, bx = _axbx(_mm(hn, wan_ref[...]), _mm(hn, wbn_ref[...]), xn)
        ax_ref[...] = ax
        bx_ref[...] = bx

    return pl.pallas_call(
        body,
        grid=(N // NTC,),
        in_specs=[_rows((NTC, HID))] * 2 + [_rows((NTC, XW))]
                 + [_rows((NTC, HID))] + [_full((HID, HID))] * 5,
        out_specs=[_rows((NTC, HID)), _rows((NTC, XW)),
                   _rows((NTC, PKW)), _rows((NTC, PKW))],
        out_shape=[jax.ShapeDtypeStruct((N, HID), _f32),
                   jax.ShapeDtypeStruct((N, XW), _f32),
                   jax.ShapeDtypeStruct((N, PKW), jnp.int32),
                   jax.ShapeDtypeStruct((N, PKW), jnp.int32)],
    )(h, hp, x, xq, wh1a, wh1b, wh2, we1an, we1bn)


def _tc_node_round(h, hp, x, xq, S, emb_p,
                   wh1a, wh1b, wh2, wm1, wm2, we1a0, we1b0):
    def body(h_ref, hp_ref, x_ref, xq_ref, s_ref, emb_ref,
             wh1a_ref, wh1b_ref, wh2_ref, wm1_ref, wm2_ref,
             wa_ref, wb_ref, ho_ref, xo_ref, ax_ref, bx_ref):
        hn = _node_common(h_ref, hp_ref, wh1a_ref, wh1b_ref, wh2_ref)
        xn = _x_common(x_ref, xq_ref)
        xo_ref[...] = xn
        mem = _mm(_silu(_mm(_silu(hn), wm1_ref[...])), wm2_ref[...])
        oh = (s_ref[...] == lax.broadcasted_iota(jnp.int32, (1, 32), 1)
              ).astype(_f32)
        hnew = _mm(oh, emb_ref[...]) + mem
        ho_ref[...] = hnew
        ax, bx = _axbx(_mm(hnew, wa_ref[...]), _mm(hnew, wb_ref[...]), xn)
        ax_ref[...] = ax
        bx_ref[...] = bx

    return pl.pallas_call(
        body,
        grid=(N // NTC,),
        in_specs=[_rows((NTC, HID))] * 2 + [_rows((NTC, XW))]
                 + [_rows((NTC, HID))] + [_rows((NTC, 1))]
                 + [_full((32, HID))] + [_full((HID, HID))] * 7,
        out_specs=[_rows((NTC, HID)), _rows((NTC, XW)),
                   _rows((NTC, PKW)), _rows((NTC, PKW))],
        out_shape=[jax.ShapeDtypeStruct((N, HID), _f32),
                   jax.ShapeDtypeStruct((N, XW), _f32),
                   jax.ShapeDtypeStruct((N, PKW), jnp.int32),
                   jax.ShapeDtypeStruct((N, PKW), jnp.int32)],
    )(h, hp, x, xq, S, emb_p, wh1a, wh1b, wh2, wm1, wm2, we1a0, we1b0)


def _tc_node_final(h, hp, wh1a, wh1b, wh2, wr1, wr2):
    def body(h_ref, hp_ref, wh1a_ref, wh1b_ref, wh2_ref,
             wr1_ref, wr2_ref, o_ref):
        hn = _node_common(h_ref, hp_ref, wh1a_ref, wh1b_ref, wh2_ref)
        o_ref[...] = _mm(_silu(_mm(_silu(hn), wr1_ref[...])), wr2_ref[...])

    return pl.pallas_call(
        body,
        grid=(N // NTC,),
        in_specs=[_rows((NTC, HID))] * 2 + [_full((HID, HID))] * 4
                 + [_full((HID, NCLS))],
        out_specs=_rows((NTC, NCLS)),
        out_shape=jax.ShapeDtypeStruct((N, NCLS), _f32),
    )(h, hp, wh1a, wh1b, wh2, wr1, wr2)


def kernel(X, S, edge_index, emb, W_rad, W_e1, W_e2, W_x, W_h1, W_h2,
           W_m1, W_m2, W_r1, W_r2):
    ru = jnp.asarray(_RU_NP)
    rv = jnp.asarray(_RV_NP)
    re = jnp.asarray(_RE_NP)
    icd = jnp.asarray(_ICD_NP)
    idc = jnp.asarray(_IDC_NP)
    offd = jnp.asarray(_OFFD_NP)

    Xp = jnp.pad(X.reshape(N, C * 3), ((0, 0), (0, XW - C * 3)))
    emb_p = jnp.pad(emb, ((0, 32 - NCLS), (0, 0)))
    S32 = S.astype(jnp.int32).reshape(N, 1)

    src = edge_index[0].astype(jnp.int32)
    dst = edge_index[1].astype(jnp.int32)
    padn = EP - E
    spread = jnp.arange(padn, dtype=jnp.int32)
    src_p = jnp.concatenate([src, spread % N])
    dst_p = jnp.concatenate([dst, spread % N])
    dst_t = jnp.concatenate([dst, N + spread % (NT - N)])
    srcg = [src_p[hh * EPH:(hh + 1) * EPH].reshape(32, CPTG, CH)
            for hh in range(2)]
    dstg = [dst_p[hh * EPH:(hh + 1) * EPH].reshape(32, CPTG, CH)
            for hh in range(2)]
    dsts = dst_t.reshape(16, CPTS, CH)

    wsym = [jnp.pad(W_rad[l][icd] + W_rad[l][idc] * offd[:, None],
                    ((0, PW - NPAIR), (0, 0))) for l in range(NL)]
    we1a = [W_e1[l, :HID] for l in range(NL)]
    we1b = [W_e1[l, HID:2 * HID] for l in range(NL)]
    we1r = [W_e1[l, 2 * HID:] for l in range(NL)]
    wxp = [jnp.pad(W_x[l], ((0, 0), (0, 16 - C))) for l in range(NL)]
    wh1a = [W_h1[l, :HID] for l in range(NL)]
    wh1b = [W_h1[l, HID:] for l in range(NL)]

    h, AX, BX = _tc_init(S32, Xp, emb_p, we1a[0], we1b[0])
    x = Xp
    logits = None
    for r in range(ROUNDS):
        for l in range(NL):
            GA1, GB1 = _sc_gather(AX, BX, srcg[0], dstg[0])
            GA2, GB2 = _sc_gather(AX, BX, srcg[1], dstg[1])
            Mv1, WX1 = _tc_edge(GA1, GB1, ru, rv, wsym[l],
                                we1r[l], W_e2[l], wxp[l], re)
            Mv2, WX2 = _tc_edge(GA2, GB2, ru, rv, wsym[l],
                                we1r[l], W_e2[l], wxp[l], re)
            Hp, Xq = _sc_scatter(Mv1, WX1, Mv2, WX2, dsts)
            hp = Hp[:N]
            xq = Xq[:N]
            last = l == NL - 1
            if not last:
                h, x, AX, BX = _tc_node_mid(h, hp, x, xq,
                                            wh1a[l], wh1b[l], W_h2[l],
                                            we1a[l + 1], we1b[l + 1])
            elif r < ROUNDS - 1:
                h, x, AX, BX = _tc_node_round(h, hp, x, xq, S32, emb_p,
                                              wh1a[l], wh1b[l], W_h2[l],
                                              W_m1, W_m2, we1a[0], we1b[0])
            else:
                logits = _tc_node_final(h, hp, wh1a[l], wh1b[l],
                                        W_h2[l], W_r1, W_r2)
    return logits


# speedup vs baseline: 2.2835x; 1.1740x over previous
import functools

import numpy as np
import jax
import jax.numpy as jnp
from jax import lax
from jax.experimental import pallas as pl
from jax.experimental.pallas import tpu as pltpu
from jax.experimental.pallas import tpu_sc as plsc

N = 10000
E = 90000
C = 14
HID = 128
NCLS = 25
NL = 3
ROUNDS = 3

XW = 48
NPAIR = C * (C + 1) // 2
PW = 128
PKW = 128

CH = 120
CPTG = 12
CPTS = 24
EPH = 32 * CH * CPTG
EP = 2 * EPH
NT = 10240
STRIPE = NT // 16

ET = 1024
NTC = 2000

_f32 = jnp.float32
_bf16 = jnp.bfloat16


def _mm(a, b):
    return lax.dot_general(a, b, (((a.ndim - 1,), (0,)), ((), ())),
                           preferred_element_type=_f32)


def _mmb(a, b):
    return lax.dot_general(a.astype(_bf16), b.astype(_bf16),
                           (((a.ndim - 1,), (0,)), ((), ())),
                           preferred_element_type=_f32)


def _silu(x):
    y = 0.5 * x
    return y + y * jnp.tanh(y)


def _np_expand_consts():
    pairs = [(c, d) for c in range(C) for d in range(c, C)]
    ru = np.zeros((XW, 3 * PW), np.float32)
    rv = np.zeros((XW, 3 * PW), np.float32)
    for p, (c, d) in enumerate(pairs):
        for i in range(3):
            ru[c * 3 + i, i * PW + p] = 1.0
            rv[d * 3 + i, i * PW + p] = 1.0
    re = np.zeros((16, XW), np.float32)
    for c in range(C):
        for i in range(3):
            re[c, c * 3 + i] = 1.0
    idx_cd = np.array([c * C + d for (c, d) in pairs], np.int32)
    idx_dc = np.array([d * C + c for (c, d) in pairs], np.int32)
    offd = np.array([1.0 if c != d else 0.0 for (c, d) in pairs], np.float32)
    return ru, rv, re, idx_cd, idx_dc, offd


_RU_NP, _RV_NP, _RE_NP, _ICD_NP, _IDC_NP, _OFFD_NP = _np_expand_consts()


@functools.cache
def _sc_mesh():
    return plsc.VectorSubcoreMesh(core_axis_name="c", subcore_axis_name="s")


def _sc_gather(AX, BX, srcg, dstg):
    @functools.partial(
        pl.kernel,
        out_type=[
            jax.ShapeDtypeStruct((EPH, PKW), jnp.int32),
            jax.ShapeDtypeStruct((EPH, PKW), jnp.int32),
        ],
        mesh=_sc_mesh(),
        scratch_types=[
            pltpu.VMEM((CPTG, CH), jnp.int32),
            pltpu.VMEM((CPTG, CH), jnp.int32),
            pltpu.VMEM((CH, PKW), jnp.int32),
            pltpu.VMEM((CH, PKW), jnp.int32),
            pltpu.VMEM((CH, PKW), jnp.int32),
            pltpu.VMEM((CH, PKW), jnp.int32),
            pltpu.SemaphoreType.DMA,
            pltpu.SemaphoreType.DMA,
            pltpu.SemaphoreType.DMA,
            pltpu.SemaphoreType.DMA,
        ],
        name="sc_gather",
    )
    def k(ax_h, bx_h, srcg_h, dstg_h, oa, ob,
          isrc, idst, ba0, bb0, ba1, bb1, gs0, gs1, ws0, ws1):
        w = lax.axis_index("c") * 16 + lax.axis_index("s")
        pltpu.sync_copy(srcg_h.at[w], isrc)
        pltpu.sync_copy(dstg_h.at[w], idst)

        @pl.loop(0, CPTG, step=2)
        def body(j0):
            g0a = pltpu.async_copy(ax_h.at[isrc.at[j0]], ba0, gs0)
            g0b = pltpu.async_copy(bx_h.at[idst.at[j0]], bb0, gs0)
            g1a = pltpu.async_copy(ax_h.at[isrc.at[j0 + 1]], ba1, gs1)
            g1b = pltpu.async_copy(bx_h.at[idst.at[j0 + 1]], bb1, gs1)
            e0 = (w * CPTG + j0) * CH
            g0a.wait(); g0b.wait()
            w0a = pltpu.async_copy(ba0, oa.at[pl.ds(e0, CH)], ws0)
            w0b = pltpu.async_copy(bb0, ob.at[pl.ds(e0, CH)], ws0)
            g1a.wait(); g1b.wait()
            w1a = pltpu.async_copy(ba1, oa.at[pl.ds(e0 + CH, CH)], ws1)
            w1b = pltpu.async_copy(bb1, ob.at[pl.ds(e0 + CH, CH)], ws1)
            w0a.wait(); w0b.wait(); w1a.wait(); w1b.wait()

    return k(AX, BX, srcg, dstg)


def _sc_scatter(Mv, WX, dsts):
    @functools.partial(
        pl.kernel,
        out_type=[
            jax.ShapeDtypeStruct((NT, HID), _f32),
            jax.ShapeDtypeStruct((NT, HID), _f32),
        ],
        mesh=_sc_mesh(),
        scratch_types=[
            pltpu.VMEM_SHARED((NT, HID), _f32),
            pltpu.VMEM((CPTS, CH), jnp.int32),
            pltpu.VMEM((CH, HID), _f32),
            pltpu.VMEM((CH, HID), _f32),
            pltpu.VMEM((64, HID), _f32),
            pltpu.SemaphoreType.DMA,
            pltpu.SemaphoreType.DMA,
            pltpu.SemaphoreType.DMA,
        ],
        name="sc_scatter",
    )
    def k(m_h, wx_h, dsts_h, oh, ox, acc, idx, b0, b1, zb, s0, s1, r0s):
        cid = lax.axis_index("c")
        sid = lax.axis_index("s")
        r0 = sid * STRIPE
        z16 = jnp.zeros((16,), _f32)

        @pl.loop(0, 64 * HID // 16)
        def zf(kk):
            zb[kk // (HID // 16), pl.ds((kk % (HID // 16)) * 16, 16)] = z16

        @pl.loop(0, STRIPE // 64)
        def zs(kk):
            pltpu.sync_copy(zb, acc.at[pl.ds(r0 + kk * 64, 64)])

        pltpu.sync_copy(dsts_h.at[sid], idx)
        plsc.subcore_barrier()

        def scatter_loop(src_h):
            def load(j, buf, sem):
                e0 = (sid * CPTS + j) * CH
                pltpu.async_copy(src_h.at[pl.ds(e0, CH)], buf, sem)

            def lwait(buf, sem):
                pltpu.make_async_copy(src_h.at[pl.ds(0, CH)], buf, sem).wait()

            load(0, b0, s0)
            load(1, b1, s1)

            @pl.loop(0, CPTS, step=2)
            def body(j0):
                lwait(b0, s0)
                pltpu.sync_copy(b0, acc.at[idx.at[j0]], add=True)

                @pl.when(j0 + 2 < CPTS)
                def _():
                    load(j0 + 2, b0, s0)

                lwait(b1, s1)
                pltpu.sync_copy(b1, acc.at[idx.at[j0 + 1]], add=True)

                @pl.when(j0 + 3 < CPTS)
                def _():
                    load(j0 + 3, b1, s1)

        @pl.when(cid == 0)
        def _():
            scatter_loop(m_h)

        @pl.when(cid == 1)
        def _():
            scatter_loop(wx_h)

        plsc.subcore_barrier()

        def readout(out_h):
            @pl.loop(0, STRIPE // 64)
            def ro(kk):
                pltpu.sync_copy(acc.at[pl.ds(r0 + kk * 64, 64)], zb)
                pltpu.sync_copy(zb, out_h.at[pl.ds(r0 + kk * 64, 64)])

        @pl.when(cid == 0)
        def _():
            readout(oh)

        @pl.when(cid == 1)
        def _():
            readout(ox)

    return k(Mv, WX, dsts)


def _full(shape):
    return pl.BlockSpec(shape, lambda i: (0,) * len(shape))


def _rows(bshape):
    return pl.BlockSpec(bshape, lambda i: (i,) + (0,) * (len(bshape) - 1))


def _pack(a, x):
    xp = jnp.concatenate([x, jnp.zeros((x.shape[0], 128 - XW), _f32)], axis=1)
    ua = lax.bitcast_convert_type(a.astype(_bf16).astype(_f32), jnp.uint32)
    ux = lax.bitcast_convert_type(xp.astype(_bf16).astype(_f32), jnp.uint32)
    word = jnp.bitwise_or(jnp.bitwise_and(ux, jnp.uint32(0xFFFF0000)),
                          lax.shift_right_logical(ua, jnp.uint32(16)))
    return lax.bitcast_convert_type(word, jnp.int32)


def _axbx(a, b, x):
    return _pack(a, x), _pack(b, x)


def _unpack(g_i32):
    u = lax.bitcast_convert_type(g_i32, jnp.uint32)
    a = lax.bitcast_convert_type(lax.shift_left(u, jnp.uint32(16)), _f32)
    x = lax.bitcast_convert_type(
        jnp.bitwise_and(u, jnp.uint32(0xFFFF0000)), _f32)
    return a, x


def _tc_init(S, Xp, emb_p, we1a0, we1b0):
    def body(s_ref, x_ref, emb_ref, wa_ref, wb_ref, h_ref, ax_ref, bx_ref):
        s = s_ref[...]
        oh = (s == lax.broadcasted_iota(jnp.int32, (1, 32), 1)).astype(_f32)
        h = _mm(oh, emb_ref[...])
        h_ref[...] = h
        ax, bx = _axbx(_mm(h, wa_ref[...]), _mm(h, wb_ref[...]), x_ref[...])
        ax_ref[...] = ax
        bx_ref[...] = bx

    return pl.pallas_call(
        body,
        grid=(N // NTC,),
        in_specs=[_rows((NTC, 1)), _rows((NTC, XW)), _full((32, HID)),
                  _full((HID, HID)), _full((HID, HID))],
        out_specs=[_rows((NTC, HID)), _rows((NTC, PKW)), _rows((NTC, PKW))],
        out_shape=[jax.ShapeDtypeStruct((N, HID), _f32),
                   jax.ShapeDtypeStruct((N, PKW), jnp.int32),
                   jax.ShapeDtypeStruct((N, PKW), jnp.int32)],
    )(S, Xp, emb_p, we1a0, we1b0)


def _tc_edge(GA, GB, ru, rv, wsym, we1r, we2, wxp, re):
    def body(ga_ref, gb_ref, ru_ref, rv_ref, wsym_ref,
             we1r_ref, we2_ref, wx_ref, re_ref, m_ref, wx_out_ref):
        a128, xs128 = _unpack(ga_ref[...])
        b128, xdst128 = _unpack(gb_ref[...])
        xd48 = xs128[:, :XW] - xdst128[:, :XW]
        xd = xd48
        p = _mmb(xd48, ru_ref[...])
        q = _mmb(xd48, rv_ref[...])
        rad = (p[:, :PW] * q[:, :PW]
               + p[:, PW:2 * PW] * q[:, PW:2 * PW]
               + p[:, 2 * PW:] * q[:, 2 * PW:])
        radn = rad / (1.0 + jnp.abs(rad))
        r = _silu(_mmb(radn, wsym_ref[...]))
        m1 = _silu(a128 + b128 + _mmb(r, we1r_ref[...]))
        m = _silu(_mmb(m1, we2_ref[...]))
        cw = jnp.tanh(_mmb(m, wx_ref[...]))
        m_ref[...] = m
        wxd = xd * _mmb(cw, re_ref[...])
        wx_out_ref[...] = jnp.concatenate(
            [wxd, jnp.zeros((ET, 112 - XW), _f32), jnp.ones((ET, 16), _f32)],
            axis=1)

    return pl.pallas_call(
        body,
        grid=(EPH // ET,),
        in_specs=[_rows((ET, PKW)), _rows((ET, PKW)),
                  _full((XW, 3 * PW)), _full((XW, 3 * PW)),
                  _full((PW, HID)), _full((HID, HID)), _full((HID, HID)),
                  _full((HID, 16)), _full((16, XW))],
        out_specs=[_rows((ET, HID)), _rows((ET, HID))],
        out_shape=[jax.ShapeDtypeStruct((EPH, HID), _f32),
                   jax.ShapeDtypeStruct((EPH, HID), _f32)],
    )(GA, GB, ru, rv, wsym, we1r, we2, wxp, re)


def _node_common(h_ref, hp1_ref, hp2_ref, wh1a_ref, wh1b_ref, wh2_ref):
    h = h_ref[...]
    hagg = hp1_ref[...] + hp2_ref[...]
    t = _silu(_mm(h, wh1a_ref[...]) + _mm(hagg, wh1b_ref[...]))
    return h + _mm(t, wh2_ref[...])


def _x_common(x_ref, xq1_ref, xq2_ref):
    xq = xq1_ref[...] + xq2_ref[...]
    deg = xq[:, 112:113]
    return x_ref[...] + xq[:, :XW] / (deg + 1.0)


def _tc_node_mid(h, hp1, hp2, x, xq1, xq2, wh1a, wh1b, wh2, we1an, we1bn):
    def body(h_ref, hp1_ref, hp2_ref, x_ref, xq1_ref, xq2_ref,
             wh1a_ref, wh1b_ref, wh2_ref, wan_ref, wbn_ref,
             ho_ref, xo_ref, ax_ref, bx_ref):
        hn = _node_common(h_ref, hp1_ref, hp2_ref, wh1a_ref, wh1b_ref,
                          wh2_ref)
        xn = _x_common(x_ref, xq1_ref, xq2_ref)
        ho_ref[...] = hn
        xo_ref[...] = xn
        ax, bx = _axbx(_mm(hn, wan_ref[...]), _mm(hn, wbn_ref[...]), xn)
        ax_ref[...] = ax
        bx_ref[...] = bx

    return pl.pallas_call(
        body,
        grid=(N // NTC,),
        in_specs=[_rows((NTC, HID))] * 3 + [_rows((NTC, XW))]
                 + [_rows((NTC, HID))] * 2 + [_full((HID, HID))] * 5,
        out_specs=[_rows((NTC, HID)), _rows((NTC, XW)),
                   _rows((NTC, PKW)), _rows((NTC, PKW))],
        out_shape=[jax.ShapeDtypeStruct((N, HID), _f32),
                   jax.ShapeDtypeStruct((N, XW), _f32),
                   jax.ShapeDtypeStruct((N, PKW), jnp.int32),
                   jax.ShapeDtypeStruct((N, PKW), jnp.int32)],
    )(h, hp1, hp2, x, xq1, xq2, wh1a, wh1b, wh2, we1an,
      we1bn)


def _tc_node_round(h, hp1, hp2, x, xq1, xq2, S, emb_p,
                   wh1a, wh1b, wh2, wm1, wm2, we1a0, we1b0):
    def body(h_ref, hp1_ref, hp2_ref, x_ref, xq1_ref, xq2_ref, s_ref,
             emb_ref, wh1a_ref, wh1b_ref, wh2_ref, wm1_ref, wm2_ref,
             wa_ref, wb_ref, ho_ref, xo_ref, ax_ref, bx_ref):
        hn = _node_common(h_ref, hp1_ref, hp2_ref, wh1a_ref, wh1b_ref,
                          wh2_ref)
        xn = _x_common(x_ref, xq1_ref, xq2_ref)
        xo_ref[...] = xn
        mem = _mm(_silu(_mm(_silu(hn), wm1_ref[...])), wm2_ref[...])
        oh = (s_ref[...] == lax.broadcasted_iota(jnp.int32, (1, 32), 1)
              ).astype(_f32)
        hnew = _mm(oh, emb_ref[...]) + mem
        ho_ref[...] = hnew
        ax, bx = _axbx(_mm(hnew, wa_ref[...]), _mm(hnew, wb_ref[...]), xn)
        ax_ref[...] = ax
        bx_ref[...] = bx

    return pl.pallas_call(
        body,
        grid=(N // NTC,),
        in_specs=[_rows((NTC, HID))] * 3 + [_rows((NTC, XW))]
                 + [_rows((NTC, HID))] * 2 + [_rows((NTC, 1))]
                 + [_full((32, HID))] + [_full((HID, HID))] * 7,
        out_specs=[_rows((NTC, HID)), _rows((NTC, XW)),
                   _rows((NTC, PKW)), _rows((NTC, PKW))],
        out_shape=[jax.ShapeDtypeStruct((N, HID), _f32),
                   jax.ShapeDtypeStruct((N, XW), _f32),
                   jax.ShapeDtypeStruct((N, PKW), jnp.int32),
                   jax.ShapeDtypeStruct((N, PKW), jnp.int32)],
    )(h, hp1, hp2, x, xq1, xq2, S, emb_p, wh1a, wh1b,
      wh2, wm1, wm2, we1a0, we1b0)


def _tc_node_final(h, hp1, hp2, wh1a, wh1b, wh2, wr1, wr2):
    def body(h_ref, hp1_ref, hp2_ref, wh1a_ref, wh1b_ref, wh2_ref,
             wr1_ref, wr2_ref, o_ref):
        hn = _node_common(h_ref, hp1_ref, hp2_ref, wh1a_ref, wh1b_ref,
                          wh2_ref)
        o_ref[...] = _mm(_silu(_mm(_silu(hn), wr1_ref[...])), wr2_ref[...])

    return pl.pallas_call(
        body,
        grid=(N // NTC,),
        in_specs=[_rows((NTC, HID))] * 3 + [_full((HID, HID))] * 4
                 + [_full((HID, NCLS))],
        out_specs=_rows((NTC, NCLS)),
        out_shape=jax.ShapeDtypeStruct((N, NCLS), _f32),
    )(h, hp1, hp2, wh1a, wh1b, wh2, wr1, wr2)


def kernel(X, S, edge_index, emb, W_rad, W_e1, W_e2, W_x, W_h1, W_h2,
           W_m1, W_m2, W_r1, W_r2):
    ru = jnp.asarray(_RU_NP)
    rv = jnp.asarray(_RV_NP)
    re = jnp.asarray(_RE_NP)
    icd = jnp.asarray(_ICD_NP)
    idc = jnp.asarray(_IDC_NP)
    offd = jnp.asarray(_OFFD_NP)

    Xp = jnp.pad(X.reshape(N, C * 3), ((0, 0), (0, XW - C * 3)))
    emb_p = jnp.pad(emb, ((0, 32 - NCLS), (0, 0)))
    S32 = S.astype(jnp.int32).reshape(N, 1)

    src = edge_index[0].astype(jnp.int32)
    dst = edge_index[1].astype(jnp.int32)
    padn = EP - E
    spread = jnp.arange(padn, dtype=jnp.int32)
    src_p = jnp.concatenate([src, spread % N])
    dst_p = jnp.concatenate([dst, spread % N])
    dst_t = jnp.concatenate([dst, N + spread % (NT - N)])
    srcg = [src_p[hh * EPH:(hh + 1) * EPH].reshape(32, CPTG, CH)
            for hh in range(2)]
    dstg = [dst_p[hh * EPH:(hh + 1) * EPH].reshape(32, CPTG, CH)
            for hh in range(2)]
    dsts = [dst_t[hh * EPH:(hh + 1) * EPH].reshape(16, CPTS, CH)
            for hh in range(2)]

    wsym = [jnp.pad(W_rad[l][icd] + W_rad[l][idc] * offd[:, None],
                    ((0, PW - NPAIR), (0, 0))) for l in range(NL)]
    we1a = [W_e1[l, :HID] for l in range(NL)]
    we1b = [W_e1[l, HID:2 * HID] for l in range(NL)]
    we1r = [W_e1[l, 2 * HID:] for l in range(NL)]
    wxp = [jnp.pad(W_x[l], ((0, 0), (0, 16 - C))) for l in range(NL)]
    wh1a = [W_h1[l, :HID] for l in range(NL)]
    wh1b = [W_h1[l, HID:] for l in range(NL)]

    h, AX, BX = _tc_init(S32, Xp, emb_p, we1a[0], we1b[0])
    x = Xp
    logits = None
    for r in range(ROUNDS):
        for l in range(NL):
            GA1, GB1 = _sc_gather(AX, BX, srcg[0], dstg[0])
            GA2, GB2 = _sc_gather(AX, BX, srcg[1], dstg[1])
            Mv1, WX1 = _tc_edge(GA1, GB1, ru, rv, wsym[l],
                                we1r[l], W_e2[l], wxp[l], re)
            Mv2, WX2 = _tc_edge(GA2, GB2, ru, rv, wsym[l],
                                we1r[l], W_e2[l], wxp[l], re)
            hp1, xq1 = _sc_scatter(Mv1, WX1, dsts[0])
            hp2, xq2 = _sc_scatter(Mv2, WX2, dsts[1])
            last = l == NL - 1
            if not last:
                h, x, AX, BX = _tc_node_mid(h, hp1, hp2, x, xq1, xq2,
                                            wh1a[l], wh1b[l], W_h2[l],
                                            we1a[l + 1], we1b[l + 1])
            elif r < ROUNDS - 1:
                h, x, AX, BX = _tc_node_round(h, hp1, hp2, x, xq1, xq2,
                                              S32, emb_p,
                                              wh1a[l], wh1b[l], W_h2[l],
                                              W_m1, W_m2, we1a[0], we1b[0])
            else:
                logits = _tc_node_final(h, hp1, hp2, wh1a[l], wh1b[l],
                                        W_h2[l], W_r1, W_r2)
    return logits


# speedup vs baseline: 2.4849x; 1.0882x over previous
import functools

import numpy as np
import jax
import jax.numpy as jnp
from jax import lax
from jax.experimental import pallas as pl
from jax.experimental.pallas import tpu as pltpu
from jax.experimental.pallas import tpu_sc as plsc

N = 10000
E = 90000
C = 14
HID = 128
NCLS = 25
NL = 3
ROUNDS = 3

XW = 48
NPAIR = C * (C + 1) // 2
PW = 128
PKW = 128

CH = 120
CPTG = 12
CPTS = 24
EPH = 32 * CH * CPTG
EP = 2 * EPH
NT = 10240
STRIPE = NT // 16

ET = 1536
NTC = 2000

_f32 = jnp.float32
_bf16 = jnp.bfloat16


def _mm(a, b):
    return lax.dot_general(a, b, (((a.ndim - 1,), (0,)), ((), ())),
                           preferred_element_type=_f32)


def _mmb(a, b):
    return lax.dot_general(a.astype(_bf16), b.astype(_bf16),
                           (((a.ndim - 1,), (0,)), ((), ())),
                           preferred_element_type=_f32)


def _silu(x):
    y = 0.5 * x
    return y + y * jnp.tanh(y)


def _np_expand_consts():
    pairs = [(c, d) for c in range(C) for d in range(c, C)]
    ru = np.zeros((XW, 3 * PW), np.float32)
    rv = np.zeros((XW, 3 * PW), np.float32)
    for p, (c, d) in enumerate(pairs):
        for i in range(3):
            ru[c * 3 + i, i * PW + p] = 1.0
            rv[d * 3 + i, i * PW + p] = 1.0
    re = np.zeros((16, XW), np.float32)
    for c in range(C):
        for i in range(3):
            re[c, c * 3 + i] = 1.0
    idx_cd = np.array([c * C + d for (c, d) in pairs], np.int32)
    idx_dc = np.array([d * C + c for (c, d) in pairs], np.int32)
    offd = np.array([1.0 if c != d else 0.0 for (c, d) in pairs], np.float32)
    return ru, rv, re, idx_cd, idx_dc, offd


_RU_NP, _RV_NP, _RE_NP, _ICD_NP, _IDC_NP, _OFFD_NP = _np_expand_consts()


@functools.cache
def _sc_mesh():
    return plsc.VectorSubcoreMesh(core_axis_name="c", subcore_axis_name="s")


def _sc_gather(AX, BX, srcg, dstg):
    @functools.partial(
        pl.kernel,
        out_type=[
            jax.ShapeDtypeStruct((EPH, PKW), jnp.int32),
            jax.ShapeDtypeStruct((EPH, PKW), jnp.int32),
        ],
        mesh=_sc_mesh(),
        scratch_types=[
            pltpu.VMEM((CPTG, CH), jnp.int32),
            pltpu.VMEM((CPTG, CH), jnp.int32),
            pltpu.VMEM((CH, PKW), jnp.int32),
            pltpu.VMEM((CH, PKW), jnp.int32),
            pltpu.VMEM((CH, PKW), jnp.int32),
            pltpu.VMEM((CH, PKW), jnp.int32),
            pltpu.SemaphoreType.DMA,
            pltpu.SemaphoreType.DMA,
            pltpu.SemaphoreType.DMA,
            pltpu.SemaphoreType.DMA,
        ],
        name="sc_gather",
    )
    def k(ax_h, bx_h, srcg_h, dstg_h, oa, ob,
          isrc, idst, ba0, bb0, ba1, bb1, gs0, gs1, ws0, ws1):
        w = lax.axis_index("c") * 16 + lax.axis_index("s")
        pltpu.sync_copy(srcg_h.at[w], isrc)
        pltpu.sync_copy(dstg_h.at[w], idst)

        @pl.loop(0, CPTG, step=2)
        def body(j0):
            g0a = pltpu.async_copy(ax_h.at[isrc.at[j0]], ba0, gs0)
            g0b = pltpu.async_copy(bx_h.at[idst.at[j0]], bb0, gs0)
            g1a = pltpu.async_copy(ax_h.at[isrc.at[j0 + 1]], ba1, gs1)
            g1b = pltpu.async_copy(bx_h.at[idst.at[j0 + 1]], bb1, gs1)
            e0 = (w * CPTG + j0) * CH
            g0a.wait(); g0b.wait()
            w0a = pltpu.async_copy(ba0, oa.at[pl.ds(e0, CH)], ws0)
            w0b = pltpu.async_copy(bb0, ob.at[pl.ds(e0, CH)], ws0)
            g1a.wait(); g1b.wait()
            w1a = pltpu.async_copy(ba1, oa.at[pl.ds(e0 + CH, CH)], ws1)
            w1b = pltpu.async_copy(bb1, ob.at[pl.ds(e0 + CH, CH)], ws1)
            w0a.wait(); w0b.wait(); w1a.wait(); w1b.wait()

    return k(AX, BX, srcg, dstg)


def _sc_scatter(Mv, WX, dsts):
    @functools.partial(
        pl.kernel,
        out_type=[
            jax.ShapeDtypeStruct((NT, HID), _f32),
            jax.ShapeDtypeStruct((NT, HID), _f32),
        ],
        mesh=_sc_mesh(),
        scratch_types=[
            pltpu.VMEM_SHARED((NT, HID), _f32),
            pltpu.VMEM((CPTS, CH), jnp.int32),
            pltpu.VMEM((CH, HID), _f32),
            pltpu.VMEM((CH, HID), _f32),
            pltpu.VMEM((64, HID), _f32),
            pltpu.SemaphoreType.DMA,
            pltpu.SemaphoreType.DMA,
            pltpu.SemaphoreType.DMA,
        ],
        name="sc_scatter",
    )
    def k(m_h, wx_h, dsts_h, oh, ox, acc, idx, b0, b1, zb, s0, s1, r0s):
        cid = lax.axis_index("c")
        sid = lax.axis_index("s")
        r0 = sid * STRIPE
        z16 = jnp.zeros((16,), _f32)

        @pl.loop(0, 64 * HID // 16)
        def zf(kk):
            zb[kk // (HID // 16), pl.ds((kk % (HID // 16)) * 16, 16)] = z16

        @pl.loop(0, STRIPE // 64)
        def zs(kk):
            pltpu.sync_copy(zb, acc.at[pl.ds(r0 + kk * 64, 64)])

        pltpu.sync_copy(dsts_h.at[sid], idx)
        plsc.subcore_barrier()

        def scatter_loop(src_h):
            def load(j, buf, sem):
                e0 = (sid * CPTS + j) * CH
                pltpu.async_copy(src_h.at[pl.ds(e0, CH)], buf, sem)

            def lwait(buf, sem):
                pltpu.make_async_copy(src_h.at[pl.ds(0, CH)], buf, sem).wait()

            load(0, b0, s0)
            load(1, b1, s1)

            @pl.loop(0, CPTS, step=2)
            def body(j0):
                lwait(b0, s0)
                pltpu.sync_copy(b0, acc.at[idx.at[j0]], add=True)

                @pl.when(j0 + 2 < CPTS)
                def _():
                    load(j0 + 2, b0, s0)

                lwait(b1, s1)
                pltpu.sync_copy(b1, acc.at[idx.at[j0 + 1]], add=True)

                @pl.when(j0 + 3 < CPTS)
                def _():
                    load(j0 + 3, b1, s1)

        @pl.when(cid == 0)
        def _():
            scatter_loop(m_h)

        @pl.when(cid == 1)
        def _():
            scatter_loop(wx_h)

        plsc.subcore_barrier()

        def readout(out_h):
            @pl.loop(0, STRIPE // 64)
            def ro(kk):
                pltpu.sync_copy(acc.at[pl.ds(r0 + kk * 64, 64)], zb)
                pltpu.sync_copy(zb, out_h.at[pl.ds(r0 + kk * 64, 64)])

        @pl.when(cid == 0)
        def _():
            readout(oh)

        @pl.when(cid == 1)
        def _():
            readout(ox)

    return k(Mv, WX, dsts)


def _full(shape):
    return pl.BlockSpec(shape, lambda i: (0,) * len(shape))


def _rows(bshape):
    return pl.BlockSpec(bshape, lambda i: (i,) + (0,) * (len(bshape) - 1))


def _pack(a, x):
    xp = jnp.concatenate([x, jnp.zeros((x.shape[0], 128 - XW), _f32)], axis=1)
    ua = lax.bitcast_convert_type(a.astype(_bf16).astype(_f32), jnp.uint32)
    ux = lax.bitcast_convert_type(xp.astype(_bf16).astype(_f32), jnp.uint32)
    word = jnp.bitwise_or(jnp.bitwise_and(ux, jnp.uint32(0xFFFF0000)),
                          lax.shift_right_logical(ua, jnp.uint32(16)))
    return lax.bitcast_convert_type(word, jnp.int32)


def _axbx(a, b, x):
    return _pack(a, x), _pack(b, x)


def _unpack(g_i32):
    u = lax.bitcast_convert_type(g_i32, jnp.uint32)
    a = lax.bitcast_convert_type(lax.shift_left(u, jnp.uint32(16)), _f32)
    x = lax.bitcast_convert_type(
        jnp.bitwise_and(u, jnp.uint32(0xFFFF0000)), _f32)
    return a, x


def _tc_init(S, Xp, emb_p, we1a0, we1b0):
    def body(s_ref, x_ref, emb_ref, wa_ref, wb_ref, h_ref, ax_ref, bx_ref):
        s = s_ref[...]
        oh = (s == lax.broadcasted_iota(jnp.int32, (1, 32), 1)).astype(_f32)
        h = _mm(oh, emb_ref[...])
        h_ref[...] = h
        ax, bx = _axbx(_mm(h, wa_ref[...]), _mm(h, wb_ref[...]), x_ref[...])
        ax_ref[...] = ax
        bx_ref[...] = bx

    return pl.pallas_call(
        body,
        grid=(N // NTC,),
        in_specs=[_rows((NTC, 1)), _rows((NTC, XW)), _full((32, HID)),
                  _full((HID, HID)), _full((HID, HID))],
        out_specs=[_rows((NTC, HID)), _rows((NTC, PKW)), _rows((NTC, PKW))],
        out_shape=[jax.ShapeDtypeStruct((N, HID), _f32),
                   jax.ShapeDtypeStruct((N, PKW), jnp.int32),
                   jax.ShapeDtypeStruct((N, PKW), jnp.int32)],
    )(S, Xp, emb_p, we1a0, we1b0)


def _tc_edge(GA, GB, ru, rv, wsym, we1r, we2, wxp, re):
    def body(ga_ref, gb_ref, ru_ref, rv_ref, wsym_ref,
             we1r_ref, we2_ref, wx_ref, re_ref, m_ref, wx_out_ref):
        a128, xs128 = _unpack(ga_ref[...])
        b128, xdst128 = _unpack(gb_ref[...])
        xd48 = xs128[:, :XW] - xdst128[:, :XW]
        xd = xd48
        p = _mmb(xd48, ru_ref[...])
        q = _mmb(xd48, rv_ref[...])
        rad = (p[:, :PW] * q[:, :PW]
               + p[:, PW:2 * PW] * q[:, PW:2 * PW]
               + p[:, 2 * PW:] * q[:, 2 * PW:])
        radn = rad / (1.0 + jnp.abs(rad))
        r = _silu(_mmb(radn, wsym_ref[...]))
        m1 = _silu(a128 + b128 + _mmb(r, we1r_ref[...]))
        m = _silu(_mmb(m1, we2_ref[...]))
        cw = jnp.tanh(_mmb(m, wx_ref[...]))
        m_ref[...] = m
        wxd = xd * _mmb(cw, re_ref[...])
        wx_out_ref[...] = jnp.concatenate(
            [wxd, jnp.zeros((ET, 112 - XW), _f32), jnp.ones((ET, 16), _f32)],
            axis=1)

    return pl.pallas_call(
        body,
        grid=(EPH // ET,),
        in_specs=[_rows((ET, PKW)), _rows((ET, PKW)),
                  _full((XW, 3 * PW)), _full((XW, 3 * PW)),
                  _full((PW, HID)), _full((HID, HID)), _full((HID, HID)),
                  _full((HID, 16)), _full((16, XW))],
        out_specs=[_rows((ET, HID)), _rows((ET, HID))],
        out_shape=[jax.ShapeDtypeStruct((EPH, HID), _f32),
                   jax.ShapeDtypeStruct((EPH, HID), _f32)],
    )(GA, GB, ru, rv, wsym, we1r, we2, wxp, re)


def _node_common(h_ref, hp1_ref, hp2_ref, wh1a_ref, wh1b_ref, wh2_ref):
    h = h_ref[...]
    hagg = hp1_ref[...] + hp2_ref[...]
    t = _silu(_mm(h, wh1a_ref[...]) + _mm(hagg, wh1b_ref[...]))
    return h + _mm(t, wh2_ref[...])


def _x_common(x_ref, xq1_ref, xq2_ref):
    xq = xq1_ref[...] + xq2_ref[...]
    deg = xq[:, 112:113]
    return x_ref[...] + xq[:, :XW] / (deg + 1.0)


def _tc_node_mid(h, hp1, hp2, x, xq1, xq2, wh1a, wh1b, wh2, we1an, we1bn):
    def body(h_ref, hp1_ref, hp2_ref, x_ref, xq1_ref, xq2_ref,
             wh1a_ref, wh1b_ref, wh2_ref, wan_ref, wbn_ref,
             ho_ref, xo_ref, ax_ref, bx_ref):
        hn = _node_common(h_ref, hp1_ref, hp2_ref, wh1a_ref, wh1b_ref,
                          wh2_ref)
        xn = _x_common(x_ref, xq1_ref, xq2_ref)
        ho_ref[...] = hn
        xo_ref[...] = xn
        ax, bx = _axbx(_mm(hn, wan_ref[...]), _mm(hn, wbn_ref[...]), xn)
        ax_ref[...] = ax
        bx_ref[...] = bx

    return pl.pallas_call(
        body,
        grid=(N // NTC,),
        in_specs=[_rows((NTC, HID))] * 3 + [_rows((NTC, XW))]
                 + [_rows((NTC, HID))] * 2 + [_full((HID, HID))] * 5,
        out_specs=[_rows((NTC, HID)), _rows((NTC, XW)),
                   _rows((NTC, PKW)), _rows((NTC, PKW))],
        out_shape=[jax.ShapeDtypeStruct((N, HID), _f32),
                   jax.ShapeDtypeStruct((N, XW), _f32),
                   jax.ShapeDtypeStruct((N, PKW), jnp.int32),
                   jax.ShapeDtypeStruct((N, PKW), jnp.int32)],
    )(h, hp1, hp2, x, xq1, xq2, wh1a, wh1b, wh2, we1an,
      we1bn)


def _tc_node_round(h, hp1, hp2, x, xq1, xq2, S, emb_p,
                   wh1a, wh1b, wh2, wm1, wm2, we1a0, we1b0):
    def body(h_ref, hp1_ref, hp2_ref, x_ref, xq1_ref, xq2_ref, s_ref,
             emb_ref, wh1a_ref, wh1b_ref, wh2_ref, wm1_ref, wm2_ref,
             wa_ref, wb_ref, ho_ref, xo_ref, ax_ref, bx_ref):
        hn = _node_common(h_ref, hp1_ref, hp2_ref, wh1a_ref, wh1b_ref,
                          wh2_ref)
        xn = _x_common(x_ref, xq1_ref, xq2_ref)
        xo_ref[...] = xn
        mem = _mm(_silu(_mm(_silu(hn), wm1_ref[...])), wm2_ref[...])
        oh = (s_ref[...] == lax.broadcasted_iota(jnp.int32, (1, 32), 1)
              ).astype(_f32)
        hnew = _mm(oh, emb_ref[...]) + mem
        ho_ref[...] = hnew
        ax, bx = _axbx(_mm(hnew, wa_ref[...]), _mm(hnew, wb_ref[...]), xn)
        ax_ref[...] = ax
        bx_ref[...] = bx

    return pl.pallas_call(
        body,
        grid=(N // NTC,),
        in_specs=[_rows((NTC, HID))] * 3 + [_rows((NTC, XW))]
                 + [_rows((NTC, HID))] * 2 + [_rows((NTC, 1))]
                 + [_full((32, HID))] + [_full((HID, HID))] * 7,
        out_specs=[_rows((NTC, HID)), _rows((NTC, XW)),
                   _rows((NTC, PKW)), _rows((NTC, PKW))],
        out_shape=[jax.ShapeDtypeStruct((N, HID), _f32),
                   jax.ShapeDtypeStruct((N, XW), _f32),
                   jax.ShapeDtypeStruct((N, PKW), jnp.int32),
                   jax.ShapeDtypeStruct((N, PKW), jnp.int32)],
    )(h, hp1, hp2, x, xq1, xq2, S, emb_p, wh1a, wh1b,
      wh2, wm1, wm2, we1a0, we1b0)


def _tc_node_final(h, hp1, hp2, wh1a, wh1b, wh2, wr1, wr2):
    def body(h_ref, hp1_ref, hp2_ref, wh1a_ref, wh1b_ref, wh2_ref,
             wr1_ref, wr2_ref, o_ref):
        hn = _node_common(h_ref, hp1_ref, hp2_ref, wh1a_ref, wh1b_ref,
                          wh2_ref)
        o_ref[...] = _mm(_silu(_mm(_silu(hn), wr1_ref[...])), wr2_ref[...])

    return pl.pallas_call(
        body,
        grid=(N // NTC,),
        in_specs=[_rows((NTC, HID))] * 3 + [_full((HID, HID))] * 4
                 + [_full((HID, NCLS))],
        out_specs=_rows((NTC, NCLS)),
        out_shape=jax.ShapeDtypeStruct((N, NCLS), _f32),
    )(h, hp1, hp2, wh1a, wh1b, wh2, wr1, wr2)


def kernel(X, S, edge_index, emb, W_rad, W_e1, W_e2, W_x, W_h1, W_h2,
           W_m1, W_m2, W_r1, W_r2):
    ru = jnp.asarray(_RU_NP).astype(_bf16)
    rv = jnp.asarray(_RV_NP).astype(_bf16)
    re = jnp.asarray(_RE_NP).astype(_bf16)
    icd = jnp.asarray(_ICD_NP)
    idc = jnp.asarray(_IDC_NP)
    offd = jnp.asarray(_OFFD_NP)

    Xp = jnp.pad(X.reshape(N, C * 3), ((0, 0), (0, XW - C * 3)))
    emb_p = jnp.pad(emb, ((0, 32 - NCLS), (0, 0)))
    S32 = S.astype(jnp.int32).reshape(N, 1)

    src = edge_index[0].astype(jnp.int32)
    dst = edge_index[1].astype(jnp.int32)
    padn = EP - E
    spread = jnp.arange(padn, dtype=jnp.int32)
    src_p = jnp.concatenate([src, spread % N])
    dst_p = jnp.concatenate([dst, spread % N])
    dst_t = jnp.concatenate([dst, N + spread % (NT - N)])
    srcg = [src_p[hh * EPH:(hh + 1) * EPH].reshape(32, CPTG, CH)
            for hh in range(2)]
    dstg = [dst_p[hh * EPH:(hh + 1) * EPH].reshape(32, CPTG, CH)
            for hh in range(2)]
    dsts = [dst_t[hh * EPH:(hh + 1) * EPH].reshape(16, CPTS, CH)
            for hh in range(2)]

    wsym = [jnp.pad(W_rad[l][icd] + W_rad[l][idc] * offd[:, None],
                    ((0, PW - NPAIR), (0, 0))).astype(_bf16)
            for l in range(NL)]
    we1a = [W_e1[l, :HID] for l in range(NL)]
    we1b = [W_e1[l, HID:2 * HID] for l in range(NL)]
    we1r = [W_e1[l, 2 * HID:].astype(_bf16) for l in range(NL)]
    wxp = [jnp.pad(W_x[l], ((0, 0), (0, 16 - C))).astype(_bf16)
           for l in range(NL)]
    wh1a = [W_h1[l, :HID] for l in range(NL)]
    wh1b = [W_h1[l, HID:] for l in range(NL)]

    h, AX, BX = _tc_init(S32, Xp, emb_p, we1a[0], we1b[0])
    x = Xp
    logits = None
    for r in range(ROUNDS):
        for l in range(NL):
            GA1, GB1 = _sc_gather(AX, BX, srcg[0], dstg[0])
            GA2, GB2 = _sc_gather(AX, BX, srcg[1], dstg[1])
            we2b = W_e2[l].astype(_bf16)
            Mv1, WX1 = _tc_edge(GA1, GB1, ru, rv, wsym[l],
                                we1r[l], we2b, wxp[l], re)
            Mv2, WX2 = _tc_edge(GA2, GB2, ru, rv, wsym[l],
                                we1r[l], we2b, wxp[l], re)
            hp1, xq1 = _sc_scatter(Mv1, WX1, dsts[0])
            hp2, xq2 = _sc_scatter(Mv2, WX2, dsts[1])
            last = l == NL - 1
            if not last:
                h, x, AX, BX = _tc_node_mid(h, hp1, hp2, x, xq1, xq2,
                                            wh1a[l], wh1b[l], W_h2[l],
                                            we1a[l + 1], we1b[l + 1])
            elif r < ROUNDS - 1:
                h, x, AX, BX = _tc_node_round(h, hp1, hp2, x, xq1, xq2,
                                              S32, emb_p,
                                              wh1a[l], wh1b[l], W_h2[l],
                                              W_m1, W_m2, we1a[0], we1b[0])
            else:
                logits = _tc_node_final(h, hp1, hp2, wh1a[l], wh1b[l],
                                        W_h2[l], W_r1, W_r2)
    return logits
